# baseline (speedup 1.0000x reference)
import functools
import jax
import jax.numpy as jnp
from jax import lax
from jax.experimental import pallas as pl
from jax.experimental.pallas import tpu as pltpu
from jax.experimental.pallas import tpu_sc as plsc

N = 10000
NUM_ITEM = 8000
NUM_USER = 2000
E = 60000
B = 1024
DX = 64

NC = 2
NS = 16
NW = NC * NS
EW = 2048
EP = NW * EW
KB = 16
ECA = 32
NKA = EW // ECA
ZR = 80
NZC = N // ZR

F32 = jnp.float32
I32 = jnp.int32


def _leaky(x):
    return jnp.where(x >= 0, x, 0.01 * x)


def _k1a_body(feat_ref, user_ref, mw_ref, mb_ref, uw_ref, ub_ref,
              lw_ref, lb_ref, id_ref, x_ref, xhat_ref):
    pid = pl.program_id(0)

    @pl.when(pid < 8)
    def _():
        x_ref[...] = jnp.tanh(
            jnp.dot(feat_ref[...], mw_ref[...], preferred_element_type=F32)
            + mb_ref[...])

    @pl.when(pid >= 8)
    def _():
        x_ref[...] = jnp.tanh(
            jnp.dot(user_ref[...], uw_ref[...], preferred_element_type=F32)
            + ub_ref[...])

    r = x_ref[...]
    nrm = jnp.sqrt(jnp.sum(r * r, axis=1, keepdims=True))
    xl = r / jnp.maximum(nrm, 1e-12)
    x_ref[...] = xl
    xhat_ref[...] = _leaky(
        jnp.dot(xl, lw_ref[...], preferred_element_type=F32)
        + lb_ref[...]) + id_ref[...]


def _k1a(feat, user, mw, mb, uw, ub, lw, lb, id_emb, L, Fm):
    rb = 1000
    return pl.pallas_call(
        _k1a_body,
        grid=(10,),
        in_specs=[
            pl.BlockSpec((rb, Fm), lambda i: (jnp.minimum(i, 7), 0)),
            pl.BlockSpec((rb, 128), lambda i: (jnp.maximum(i - 8, 0), 0)),
            pl.BlockSpec((Fm, L), lambda i: (0, 0)),
            pl.BlockSpec((1, L), lambda i: (0, 0)),
            pl.BlockSpec((128, L), lambda i: (0, 0)),
            pl.BlockSpec((1, L), lambda i: (0, 0)),
            pl.BlockSpec((L, DX), lambda i: (0, 0)),
            pl.BlockSpec((1, DX), lambda i: (0, 0)),
            pl.BlockSpec((rb, DX), lambda i: (i, 0)),
        ],
        out_specs=[
            pl.BlockSpec((rb, L), lambda i: (i, 0)),
            pl.BlockSpec((rb, DX), lambda i: (i, 0)),
        ],
        out_shape=[
            jax.ShapeDtypeStruct((N, L), F32),
            jax.ShapeDtypeStruct((N, DX), F32),
        ],
    )(feat, user, mw, mb, uw, ub, lw, lb, id_emb)


def _k1b_body(nsub, x_ref, cw_ref, xw3_ref, xw2_ref):
    t = jnp.dot(x_ref[...], cw_ref[...], preferred_element_type=F32)
    for u in range(nsub):
        xw3_ref[u] = t[:, u * 64:(u + 1) * 64]
    xw2_ref[...] = t


def _k1b(x, convp, L, Lp, CWT):
    rb = 1000
    nch = Lp // CWT
    nsub = CWT // 64
    return pl.pallas_call(
        functools.partial(_k1b_body, nsub),
        grid=(10, nch),
        in_specs=[
            pl.BlockSpec((rb, L), lambda i, j: (i, 0)),
            pl.BlockSpec((L, CWT), lambda i, j: (0, j)),
        ],
        out_specs=[
            pl.BlockSpec((nsub, rb, 64), lambda i, j: (j, i, 0)),
            pl.BlockSpec((rb, CWT), lambda i, j: (i, j)),
        ],
        out_shape=[
            jax.ShapeDtypeStruct((Lp // 64, N, 64), F32),
            jax.ShapeDtypeStruct((N, Lp), F32),
        ],
    )(x, convp)


def _sca_body(Lp, xw2, srcm, dstm, spart, wout,
              idx_s, idx_d, rows_s, rows_d, wbuf, s_loc):
    cid = lax.axis_index("c")
    sid = lax.axis_index("s")
    wid = sid * NC + cid
    iota = lax.iota(I32, 16)

    pltpu.sync_copy(srcm.at[pl.ds(wid * KB, KB)], idx_s)
    pltpu.sync_copy(dstm.at[pl.ds(wid * KB, KB)], idx_d)

    def zero_body(i, _):
        s_loc[pl.ds(i * 16, 16)] = jnp.zeros((16,), F32)
        return 0
    lax.fori_loop(0, N // 16, zero_body, 0)

    def k_body(k, _):
        kb = k // 4
        off = (k % 4) * ECA
        pltpu.sync_copy(xw2.at[idx_s.at[kb, pl.ds(off, ECA)]], rows_s)
        pltpu.sync_copy(xw2.at[idx_d.at[kb, pl.ds(off, ECA)]], rows_d)

        def col_body(j, acc):
            a0, a1 = acc
            jv = jnp.full((16,), j, I32)
            s0 = plsc.load_gather(rows_s, [iota, jv])
            d0 = plsc.load_gather(rows_d, [iota, jv])
            s1 = plsc.load_gather(rows_s, [iota + 16, jv])
            d1 = plsc.load_gather(rows_d, [iota + 16, jv])
            return (a0 + s0 * d0, a1 + s1 * d1)

        z = jnp.zeros((16,), F32)
        a0, a1 = lax.fori_loop(0, Lp, col_body, (z, z), unroll=8)

        for g, inner in ((0, a0), (1, a1)):
            eid = wid * EW + k * ECA + g * 16 + iota
            mask = eid < E
            ev = jnp.where(mask, jnp.exp(inner), 0.0)
            gate = 1.0 / (1.0 + jnp.exp(-inner))
            w = jnp.where(mask, ev * gate, 0.0)
            wbuf[kb, pl.ds(off + g * 16, 16)] = w
            dst16 = idx_d[kb, pl.ds(off + g * 16, 16)]
            plsc.addupdate_scatter(s_loc, [dst16], ev)
        return 0

    lax.fori_loop(0, NKA, k_body, 0)
    pltpu.sync_copy(s_loc, spart.at[pl.ds(wid * N, N)])
    pltpu.sync_copy(wbuf, wout.at[pl.ds(wid * KB, KB)])


def _sca(xw2, srcm, dstm, Lp):
    mesh = plsc.VectorSubcoreMesh(core_axis_name="c", subcore_axis_name="s", num_cores=NC, num_subcores=NS)
    return pl.kernel(
        functools.partial(_sca_body, Lp),
        out_type=[
            jax.ShapeDtypeStruct((NW * N,), F32),
            jax.ShapeDtypeStruct((NW * KB, 128), F32),
        ],
        mesh=mesh,
        compiler_params=pltpu.CompilerParams(use_tc_tiling_on_sc=False, needs_layout_passes=False),
        scratch_types=[
            pltpu.VMEM((KB, 128), I32),
            pltpu.VMEM((KB, 128), I32),
            pltpu.VMEM((ECA, Lp), F32),
            pltpu.VMEM((ECA, Lp), F32),
            pltpu.VMEM((KB, 128), F32),
            pltpu.VMEM((N,), F32),
        ],
    )(xw2, srcm, dstm)


def _k2s_body(*refs):
    ins = refs[:4]
    outs = refs[4:]
    for i_ref, o_ref in zip(ins, outs):
        o_ref[...] = jnp.sum(i_ref[...], axis=0, keepdims=True)


def _k2s(sparts):
    return pl.pallas_call(
        _k2s_body,
        in_specs=[pl.BlockSpec((NW, N), lambda: (0, 0))] * 4,
        out_specs=[pl.BlockSpec((1, N), lambda: (0, 0))] * 4,
        out_shape=[jax.ShapeDtypeStruct((1, N), F32)] * 4,
    )(*sparts)


def _scb_body(nch, CW, xw3, srcm, dstm, wm, sm, hpart,
              idx_s, idx_d, wvm, alpha, s_vm, rows, scaled, zbuf, hsh):
    cid = lax.axis_index("c")
    sid = lax.axis_index("s")
    wid = sid * NC + cid

    pltpu.sync_copy(srcm.at[pl.ds(wid * KB, KB)], idx_s)
    pltpu.sync_copy(dstm.at[pl.ds(wid * KB, KB)], idx_d)
    pltpu.sync_copy(wm.at[pl.ds(wid * KB, KB)], wvm)
    pltpu.sync_copy(sm.at[0], s_vm)

    def zb2_body(i, _):
        r = i // (CW // 16)
        c = (i % (CW // 16)) * 16
        zbuf[r, pl.ds(c, 16)] = jnp.zeros((16,), F32)
        return 0
    lax.fori_loop(0, ZR * (CW // 16), zb2_body, 0, unroll=8)

    def al_body(i, _):
        kb = i // 8
        off = (i % 8) * 16
        dst16 = idx_d[kb, pl.ds(off, 16)]
        sv = plsc.load_gather(s_vm, [dst16])
        w16 = wvm[kb, pl.ds(off, 16)]
        alpha[kb, pl.ds(off, 16)] = w16 / (sv + 1e-16)
        return 0
    lax.fori_loop(0, KB * 8, al_body, 0)

    def chunk_body(c, _):
        def z_body(j, _):
            i = sid + j * NS

            @pl.when(i < NZC)
            def _():
                pltpu.sync_copy(zbuf, hsh.at[pl.ds(i * ZR, ZR)])
            return 0
        lax.fori_loop(0, (NZC + NS - 1) // NS, z_body, 0)
        plsc.subcore_barrier()

        def k_body(kb, _):
            pltpu.sync_copy(xw3.at[c].at[idx_s.at[kb]], rows)

            def e_body(e, _):
                sp = plsc.load_gather(
                    alpha, [jnp.full((16,), kb, I32),
                            jnp.full((16,), e, I32)])
                for j in range(CW // 16):
                    scaled[e, pl.ds(j * 16, 16)] = (
                        rows[e, pl.ds(j * 16, 16)] * sp)
                return 0
            lax.fori_loop(0, 128, e_body, 0, unroll=4)
            pltpu.sync_copy(scaled, hsh.at[idx_d.at[kb]], add=True)
            return 0
        lax.fori_loop(0, KB, k_body, 0)
        plsc.subcore_barrier()

        def o_body(j, _):
            i = sid + j * NS

            @pl.when(i < NZC)
            def _():
                pltpu.sync_copy(hsh.at[pl.ds(i * ZR, ZR)],
                                hpart.at[cid, c, pl.ds(i * ZR, ZR)])
            return 0
        lax.fori_loop(0, (NZC + NS - 1) // NS, o_body, 0)
        plsc.subcore_barrier()
        return 0

    lax.fori_loop(0, nch, chunk_body, 0)


def _scb(xw3, srcm, dstm, wm, s2, nch, CW):
    mesh = plsc.VectorSubcoreMesh(core_axis_name="c", subcore_axis_name="s", num_cores=NC, num_subcores=NS)
    return pl.kernel(
        functools.partial(_scb_body, nch, CW),
        out_type=jax.ShapeDtypeStruct((NC, nch, N, CW), F32),
        mesh=mesh,
        compiler_params=pltpu.CompilerParams(use_tc_tiling_on_sc=False, needs_layout_passes=False),
        scratch_types=[
            pltpu.VMEM((KB, 128), I32),
            pltpu.VMEM((KB, 128), I32),
            pltpu.VMEM((KB, 128), F32),
            pltpu.VMEM((KB, 128), F32),
            pltpu.VMEM((N,), F32),
            pltpu.VMEM((128, CW), F32),
            pltpu.VMEM((128, CW), F32),
            pltpu.VMEM((ZR, CW), F32),
            pltpu.VMEM_SHARED((N, CW), F32),
        ],
    )(xw3, srcm, dstm, wm, s2)


def _k3_body(nchs, CWs, *refs):
    hps = refs[0:4]
    xhs = refs[4:8]
    g1ws = refs[8:12]
    g1bs = refs[12:16]
    repms = refs[16:20]
    rep = refs[20]
    acc = None
    for m in range(4):
        hp = hps[m][...]
        h = hp[0] + hp[1]
        cw = CWs[m]
        mm = None
        for c in range(nchs[m]):
            part = jnp.dot(_leaky(h[c]),
                           g1ws[m][pl.ds(c * cw, cw), :],
                           preferred_element_type=F32)
            mm = part if mm is None else mm + part
        o = _leaky(mm + g1bs[m][...] + xhs[m][...])
        repms[m][...] = o
        acc = o if acc is None else acc + o
    rep[...] = acc * 0.25


def _k3(hparts, xhats, g1ws, g1bs, nchs, CWs, Lps):
    rb = 400
    in_specs = []
    for m in range(4):
        in_specs.append(pl.BlockSpec((NC, nchs[m], rb, CWs[m]),
                                     lambda i: (0, 0, i, 0)))
    for m in range(4):
        in_specs.append(pl.BlockSpec((rb, DX), lambda i: (i, 0)))
    for m in range(4):
        in_specs.append(pl.BlockSpec((Lps[m], DX), lambda i: (0, 0)))
    for m in range(4):
        in_specs.append(pl.BlockSpec((1, DX), lambda i: (0, 0)))
    return pl.pallas_call(
        functools.partial(_k3_body, nchs, CWs),
        grid=(25,),
        in_specs=in_specs,
        out_specs=[pl.BlockSpec((rb, DX), lambda i: (i, 0))] * 5,
        out_shape=[jax.ShapeDtypeStruct((N, DX), F32)] * 5,
    )(*hparts, *xhats, *g1ws, *g1bs)


def _sct_body(rep, rv, rt, rp, rtr, unm, pim, nim,
              user_o, pos_o, neg_o, kpart,
              idxb, rows, kp_vm):
    cid = lax.axis_index("c")
    sid = lax.axis_index("s")
    wid = sid * NC + cid
    bpw = B // NW

    def gather_out(idx_hbm, out_hbm):
        pltpu.sync_copy(idx_hbm, idxb)
        pltpu.sync_copy(rep.at[idxb.at[wid]], rows)
        pltpu.sync_copy(rows, out_hbm.at[pl.ds(wid * bpw, bpw)])

    gather_out(unm, user_o)
    gather_out(nim, neg_o)
    gather_out(pim, pos_o)

    for m, tab in enumerate((rv, rt, rp, rtr)):
        pltpu.sync_copy(tab.at[idxb.at[wid]], rows)
        for j in range(DX // 16):
            def acc_body(e, a):
                return a + rows[e, pl.ds(j * 16, 16)]
            a = lax.fori_loop(0, bpw, acc_body, jnp.zeros((16,), F32),
                              unroll=8)
            kp_vm[m, pl.ds(j * 16, 16)] = a
    for m in range(4, 8):
        for j in range(DX // 16):
            kp_vm[m, pl.ds(j * 16, 16)] = jnp.zeros((16,), F32)
    pltpu.sync_copy(kp_vm, kpart.at[wid])


def _sct(rep, repms, unm, pim, nim):
    mesh = plsc.VectorSubcoreMesh(core_axis_name="c", subcore_axis_name="s", num_cores=NC, num_subcores=NS)
    bpw = B // NW
    return pl.kernel(
        _sct_body,
        out_type=[
            jax.ShapeDtypeStruct((B, DX), F32),
            jax.ShapeDtypeStruct((B, DX), F32),
            jax.ShapeDtypeStruct((B, DX), F32),
            jax.ShapeDtypeStruct((NW, 8, DX), F32),
        ],
        mesh=mesh,
        compiler_params=pltpu.CompilerParams(use_tc_tiling_on_sc=False, needs_layout_passes=False),
        scratch_types=[
            pltpu.VMEM((NW, bpw), I32),
            pltpu.VMEM((bpw, DX), F32),
            pltpu.VMEM((8, DX), F32),
        ],
    )(rep, *repms, unm, pim, nim)


def _ktail_body(u_ref, p_ref, n_ref, kp_ref, qt_ref, kt_ref, vt_ref,
                w1a_ref, w1b_ref, b1_ref, w2_ref, b2_ref,
                pos_ref, neg_ref, price_ref):
    K8 = jnp.sum(kp_ref[...], axis=0) * (1.0 / B)
    Kp = jnp.dot(K8, kt_ref[...], preferred_element_type=F32)
    Vp = jnp.dot(K8, vt_ref[...], preferred_element_type=F32)
    Q = jnp.dot(u_ref[...], qt_ref[...], preferred_element_type=F32)
    logits = lax.dot_general(Q, Kp, (((1,), (1,)), ((), ())),
                             preferred_element_type=F32) * (1.0 / 8.0)
    col = lax.broadcasted_iota(I32, (B, 8), 1)
    logits = jnp.where(col < 4, logits, -1e30)
    mx = jnp.max(logits, axis=1, keepdims=True)
    e = jnp.exp(logits - mx)
    att_w = e / jnp.sum(e, axis=1, keepdims=True)
    att = jnp.dot(att_w, Vp, preferred_element_type=F32)
    pos_t = p_ref[...]
    pos_ref[...] = jnp.sum(att * pos_t, axis=1, keepdims=True)
    neg_ref[...] = jnp.sum(att * n_ref[...], axis=1, keepdims=True)
    hid = _leaky(jnp.dot(att, w1a_ref[...], preferred_element_type=F32)
                 + jnp.dot(pos_t, w1b_ref[...], preferred_element_type=F32)
                 + b1_ref[...])
    pr = jnp.dot(hid, w2_ref[...], preferred_element_type=F32) + b2_ref[...]
    price_ref[...] = 1.0 / (1.0 + jnp.exp(-pr[:, 0:1]))


def _ktail(user_t, pos_t, neg_t, kpart, qT, kT, vT, w1a, w1b, b1, w2p, b2p):
    full = lambda *s: pl.BlockSpec(s, lambda: tuple(0 for _ in s))
    return pl.pallas_call(
        _ktail_body,
        in_specs=[
            full(B, DX), full(B, DX), full(B, DX), full(NW, 8, DX),
            full(DX, DX), full(DX, DX), full(DX, DX),
            full(DX, DX), full(DX, DX), full(1, DX),
            full(DX, 128), full(1, 128),
        ],
        out_specs=[full(B, 1), full(B, 1), full(B, 1)],
        out_shape=[
            jax.ShapeDtypeStruct((B, 1), F32),
            jax.ShapeDtypeStruct((B, 1), F32),
            jax.ShapeDtypeStruct((B, 1), F32),
        ],
    )(user_t, pos_t, neg_t, kpart, qT, kT, vT, w1a, w1b, b1, w2p, b2p)


def kernel(v_feat, t_feat, p_feat, tr_feat, user_feat, edge_index,
           v_mlp_w, v_mlp_b, v_umlp_w, v_umlp_b, v_conv_w, v_lin1_w,
           v_lin1_b, v_g1_w, v_g1_b,
           t_mlp_w, t_mlp_b, t_umlp_w, t_umlp_b, t_conv_w, t_lin1_w,
           t_lin1_b, t_g1_w, t_g1_b,
           p_mlp_w, p_mlp_b, p_umlp_w, p_umlp_b, p_conv_w, p_lin1_w,
           p_lin1_b, p_g1_w, p_g1_b,
           tr_mlp_w, tr_mlp_b, tr_umlp_w, tr_umlp_b, tr_conv_w, tr_lin1_w,
           tr_lin1_b, tr_g1_w, tr_g1_b,
           id_emb, q_w, k_w, v_w, price_w1, price_b1, price_w2, price_b2,
           user_nodes, pos_items, neg_items):
    mods = {
        "v": (v_feat, v_mlp_w, v_mlp_b, v_umlp_w, v_umlp_b, v_conv_w,
              v_lin1_w, v_lin1_b, v_g1_w, v_g1_b, 128, 1024, 1024, 128),
        "t": (t_feat, t_mlp_w, t_mlp_b, t_umlp_w, t_umlp_b, t_conv_w,
              t_lin1_w, t_lin1_b, t_g1_w, t_g1_b, 128, 1500, 1536, 128),
        "p": (p_feat, p_mlp_w, p_mlp_b, p_umlp_w, p_umlp_b, p_conv_w,
              p_lin1_w, p_lin1_b, p_g1_w, p_g1_b, 32, 64, 64, 64),
        "tr": (tr_feat, tr_mlp_w, tr_mlp_b, tr_umlp_w, tr_umlp_b, tr_conv_w,
               tr_lin1_w, tr_lin1_b, tr_g1_w, tr_g1_b, 32, 64, 64, 64),
    }

    src = edge_index[0]
    dst = edge_index[1]
    padi = jnp.zeros((EP - E,), I32)
    srcm = jnp.concatenate([src, padi]).reshape(NW * KB, 128)
    dstm = jnp.concatenate([dst, padi]).reshape(NW * KB, 128)

    hparts, xhats, g1ws, g1bs, nchs, CWs, Lps = [], [], [], [], [], [], []
    sparts = []
    scb_args = []
    for name, (feat, mw, mb, uw, ub, cw, lw, lb, g1w, g1b,
               Fm, L, Lp, CW) in mods.items():
        nch = Lp // 64
        x, xhat = _k1a(feat, user_feat, mw, mb.reshape(1, L), uw,
                       ub.reshape(1, L), lw, lb.reshape(1, DX), id_emb, L, Fm)
        convp = cw if Lp == L else jnp.pad(cw, ((0, 0), (0, Lp - L)))
        xw3, xw2 = _k1b(x, convp, L, Lp, CW)
        CW = 64
        spart, wout = _sca(xw2, srcm, dstm, Lp)
        sparts.append(spart.reshape(NW, N))
        scb_args.append((xw3, wout, nch, CW))
        xhats.append(xhat)
        g1ws.append(g1w if Lp == L else jnp.pad(g1w, ((0, Lp - L), (0, 0))))
        g1bs.append(g1b.reshape(1, DX))
        nchs.append(nch)
        CWs.append(CW)
        Lps.append(Lp)

    s2s = _k2s(sparts)
    for (xw3, wout, nch, CW), s2 in zip(scb_args, s2s):
        hparts.append(_scb(xw3, srcm, dstm, wout,
                           s2, nch, CW))

    repm_v, repm_t, repm_p, repm_tr, rep = _k3(
        hparts, xhats, g1ws, g1bs, nchs, CWs, Lps)

    bpw = B // NW
    unm = user_nodes.reshape(NW, bpw)
    pim = pos_items.reshape(NW, bpw)
    nim = neg_items.reshape(NW, bpw)
    user_t, pos_t, neg_t, kpart = _sct(
        rep, (repm_v, repm_t, repm_p, repm_tr), unm, pim, nim)

    w2p = jnp.pad(price_w2, ((0, 0), (0, 127)))
    b2p = jnp.pad(price_b2.reshape(1, 1), ((0, 0), (0, 127)))
    pos_s, neg_s, price = _ktail(
        user_t, pos_t, neg_t, kpart, q_w.T, k_w.T, v_w.T,
        price_w1[:DX], price_w1[DX:], price_b1.reshape(1, DX), w2p, b2p)

    return (pos_s.reshape(B), neg_s.reshape(B), rep, price)


# speedup vs baseline: 1.3431x; 1.3431x over previous
import functools
import jax
import jax.numpy as jnp
from jax import lax
from jax.experimental import pallas as pl
from jax.experimental.pallas import tpu as pltpu
from jax.experimental.pallas import tpu_sc as plsc

N = 10000
NUM_ITEM = 8000
NUM_USER = 2000
E = 60000
B = 1024
DX = 64

NC = 2
NS = 16
NW = NC * NS
EW = 2048
EP = NW * EW
KB = 16
ECA = 32
NKA = EW // ECA
ZR = 80
NZC = N // ZR
EH = E // 2
EWA = 1024
EPA = NW * EWA
KBA = EWA // 128
NKA2 = EWA // ECA

F32 = jnp.float32
I32 = jnp.int32


def _leaky(x):
    return jnp.where(x >= 0, x, 0.01 * x)


def _k1a_body(feat_ref, user_ref, mw_ref, mb_ref, uw_ref, ub_ref,
              lw_ref, lb_ref, id_ref, x_ref, xhat_ref):
    pid = pl.program_id(0)

    @pl.when(pid < 8)
    def _():
        x_ref[...] = jnp.tanh(
            jnp.dot(feat_ref[...], mw_ref[...], preferred_element_type=F32)
            + mb_ref[...])

    @pl.when(pid >= 8)
    def _():
        x_ref[...] = jnp.tanh(
            jnp.dot(user_ref[...], uw_ref[...], preferred_element_type=F32)
            + ub_ref[...])

    r = x_ref[...]
    nrm = jnp.sqrt(jnp.sum(r * r, axis=1, keepdims=True))
    xl = r / jnp.maximum(nrm, 1e-12)
    x_ref[...] = xl
    xhat_ref[...] = _leaky(
        jnp.dot(xl, lw_ref[...], preferred_element_type=F32)
        + lb_ref[...]) + id_ref[...]


def _k1a(feat, user, mw, mb, uw, ub, lw, lb, id_emb, L, Fm):
    rb = 1000
    return pl.pallas_call(
        _k1a_body,
        grid=(10,),
        in_specs=[
            pl.BlockSpec((rb, Fm), lambda i: (jnp.minimum(i, 7), 0)),
            pl.BlockSpec((rb, 128), lambda i: (jnp.maximum(i - 8, 0), 0)),
            pl.BlockSpec((Fm, L), lambda i: (0, 0)),
            pl.BlockSpec((1, L), lambda i: (0, 0)),
            pl.BlockSpec((128, L), lambda i: (0, 0)),
            pl.BlockSpec((1, L), lambda i: (0, 0)),
            pl.BlockSpec((L, DX), lambda i: (0, 0)),
            pl.BlockSpec((1, DX), lambda i: (0, 0)),
            pl.BlockSpec((rb, DX), lambda i: (i, 0)),
        ],
        out_specs=[
            pl.BlockSpec((rb, L), lambda i: (i, 0)),
            pl.BlockSpec((rb, DX), lambda i: (i, 0)),
        ],
        out_shape=[
            jax.ShapeDtypeStruct((N, L), F32),
            jax.ShapeDtypeStruct((N, DX), F32),
        ],
    )(feat, user, mw, mb, uw, ub, lw, lb, id_emb)


def _k1b_body(nsub, x_ref, cw_ref, xw3_ref, xw2_ref):
    t = jnp.dot(x_ref[...], cw_ref[...], preferred_element_type=F32)
    for u in range(nsub):
        xw3_ref[u] = t[:, u * 64:(u + 1) * 64]
    xw2_ref[...] = t


def _k1b(x, convp, L, Lp, CWT):
    rb = 1000
    nch = Lp // CWT
    nsub = CWT // 64
    return pl.pallas_call(
        functools.partial(_k1b_body, nsub),
        grid=(10, nch),
        in_specs=[
            pl.BlockSpec((rb, L), lambda i, j: (i, 0)),
            pl.BlockSpec((L, CWT), lambda i, j: (0, j)),
        ],
        out_specs=[
            pl.BlockSpec((nsub, rb, 64), lambda i, j: (j, i, 0)),
            pl.BlockSpec((rb, CWT), lambda i, j: (i, j)),
        ],
        out_shape=[
            jax.ShapeDtypeStruct((Lp // 64, N, 64), F32),
            jax.ShapeDtypeStruct((N, Lp), F32),
        ],
    )(x, convp)


def _sca_body(Lp, xw2, srcm, dstm, spart, wout,
              idx_s, idx_d, rows_s, rows_d, wbuf, s_loc):
    cid = lax.axis_index("c")
    sid = lax.axis_index("s")
    wid = sid * NC + cid
    iota = lax.iota(I32, 16)

    pltpu.sync_copy(srcm.at[pl.ds(wid * KBA, KBA)], idx_s)
    pltpu.sync_copy(dstm.at[pl.ds(wid * KBA, KBA)], idx_d)

    def zero_body(i, _):
        s_loc[pl.ds(i * 16, 16)] = jnp.zeros((16,), F32)
        return 0
    lax.fori_loop(0, N // 16, zero_body, 0)

    def k_body(k, _):
        kb = k // 4
        off = (k % 4) * ECA
        pltpu.sync_copy(xw2.at[idx_s.at[kb, pl.ds(off, ECA)]], rows_s)
        pltpu.sync_copy(xw2.at[idx_d.at[kb, pl.ds(off, ECA)]], rows_d)

        def col_body(j, acc):
            a0, a1 = acc
            jv = jnp.full((16,), j, I32)
            s0 = plsc.load_gather(rows_s, [iota, jv])
            d0 = plsc.load_gather(rows_d, [iota, jv])
            s1 = plsc.load_gather(rows_s, [iota + 16, jv])
            d1 = plsc.load_gather(rows_d, [iota + 16, jv])
            return (a0 + s0 * d0, a1 + s1 * d1)

        z = jnp.zeros((16,), F32)
        a0, a1 = lax.fori_loop(0, Lp, col_body, (z, z), unroll=8)

        for g, inner in ((0, a0), (1, a1)):
            eid = wid * EWA + k * ECA + g * 16 + iota
            mask = eid < EH
            ev = jnp.where(mask, jnp.exp(inner), 0.0)
            gate = 1.0 / (1.0 + jnp.exp(-inner))
            w = jnp.where(mask, ev * gate, 0.0)
            wbuf[kb, pl.ds(off + g * 16, 16)] = w
            dst16 = idx_d[kb, pl.ds(off + g * 16, 16)]
            src16 = idx_s[kb, pl.ds(off + g * 16, 16)]
            plsc.addupdate_scatter(s_loc, [dst16], ev)
            plsc.addupdate_scatter(s_loc, [src16], ev)
        return 0

    lax.fori_loop(0, NKA2, k_body, 0)
    pltpu.sync_copy(s_loc, spart.at[pl.ds(wid * N, N)])
    pltpu.sync_copy(wbuf, wout.at[pl.ds(wid * KBA, KBA)])


def _sca(xw2, srcm, dstm, Lp):
    mesh = plsc.VectorSubcoreMesh(core_axis_name="c", subcore_axis_name="s", num_cores=NC, num_subcores=NS)
    return pl.kernel(
        functools.partial(_sca_body, Lp),
        out_type=[
            jax.ShapeDtypeStruct((NW * N,), F32),
            jax.ShapeDtypeStruct((NW * KBA, 128), F32),
        ],
        mesh=mesh,
        compiler_params=pltpu.CompilerParams(use_tc_tiling_on_sc=False, needs_layout_passes=False),
        scratch_types=[
            pltpu.VMEM((KBA, 128), I32),
            pltpu.VMEM((KBA, 128), I32),
            pltpu.VMEM((ECA, Lp), F32),
            pltpu.VMEM((ECA, Lp), F32),
            pltpu.VMEM((KBA, 128), F32),
            pltpu.VMEM((N,), F32),
        ],
    )(xw2, srcm, dstm)


def _k2s_body(*refs):
    ins = refs[:4]
    outs = refs[4:]
    for i_ref, o_ref in zip(ins, outs):
        o_ref[...] = jnp.sum(i_ref[...], axis=0, keepdims=True)


def _k2s(sparts):
    return pl.pallas_call(
        _k2s_body,
        in_specs=[pl.BlockSpec((NW, N), lambda: (0, 0))] * 4,
        out_specs=[pl.BlockSpec((1, N), lambda: (0, 0))] * 4,
        out_shape=[jax.ShapeDtypeStruct((1, N), F32)] * 4,
    )(*sparts)


def _scb_body(nch, CW, xw3, srcm, dstm, wm, sm, hpart,
              idx_s, idx_d, wvm, alpha, s_vm, rows, scaled, zbuf, hsh):
    cid = lax.axis_index("c")
    sid = lax.axis_index("s")
    wid = sid * NC + cid

    iota = lax.iota(I32, 16)
    pltpu.sync_copy(srcm.at[pl.ds(wid * KB, KB)], idx_s)
    pltpu.sync_copy(dstm.at[pl.ds(wid * KB, KB)], idx_d)
    pltpu.sync_copy(wm, wvm)
    pltpu.sync_copy(sm.at[0], s_vm)

    def zb2_body(i, _):
        r = i // (CW // 16)
        c = (i % (CW // 16)) * 16
        zbuf[r, pl.ds(c, 16)] = jnp.zeros((16,), F32)
        return 0
    lax.fori_loop(0, ZR * (CW // 16), zb2_body, 0, unroll=8)

    def al_body(i, _):
        kb = i // 8
        off = (i % 8) * 16
        eid = wid * EW + i * 16 + iota
        mask = eid < E
        me = jnp.where(eid < EH, eid, eid - EH)
        me = jnp.where(mask, me, 0)
        dst16 = idx_d[kb, pl.ds(off, 16)]
        sv = plsc.load_gather(s_vm, [dst16])
        w16 = plsc.load_gather(wvm, [me // 128, me % 128])
        a16 = jnp.where(mask, w16 / (sv + 1e-16), 0.0)
        alpha[kb, pl.ds(off, 16)] = a16
        return 0
    lax.fori_loop(0, KB * 8, al_body, 0)

    def chunk_body(c, _):
        def z_body(j, _):
            i = sid + j * NS

            @pl.when(i < NZC)
            def _():
                pltpu.sync_copy(zbuf, hsh.at[pl.ds(i * ZR, ZR)])
            return 0
        lax.fori_loop(0, (NZC + NS - 1) // NS, z_body, 0)
        plsc.subcore_barrier()

        def k_body(kb, _):
            pltpu.sync_copy(xw3.at[c].at[idx_s.at[kb]], rows)

            def e_body(e, _):
                sp = plsc.load_gather(
                    alpha, [jnp.full((16,), kb, I32),
                            jnp.full((16,), e, I32)])
                for j in range(CW // 16):
                    scaled[e, pl.ds(j * 16, 16)] = (
                        rows[e, pl.ds(j * 16, 16)] * sp)
                return 0
            lax.fori_loop(0, 128, e_body, 0, unroll=4)
            pltpu.sync_copy(scaled, hsh.at[idx_d.at[kb]], add=True)
            return 0
        lax.fori_loop(0, KB, k_body, 0)
        plsc.subcore_barrier()

        def o_body(j, _):
            i = sid + j * NS

            @pl.when(i < NZC)
            def _():
                pltpu.sync_copy(hsh.at[pl.ds(i * ZR, ZR)],
                                hpart.at[cid, c, pl.ds(i * ZR, ZR)])
            return 0
        lax.fori_loop(0, (NZC + NS - 1) // NS, o_body, 0)
        plsc.subcore_barrier()
        return 0

    lax.fori_loop(0, nch, chunk_body, 0)


def _scb(xw3, srcm, dstm, wm, s2, nch, CW):
    mesh = plsc.VectorSubcoreMesh(core_axis_name="c", subcore_axis_name="s", num_cores=NC, num_subcores=NS)
    return pl.kernel(
        functools.partial(_scb_body, nch, CW),
        out_type=jax.ShapeDtypeStruct((NC, nch, N, CW), F32),
        mesh=mesh,
        compiler_params=pltpu.CompilerParams(use_tc_tiling_on_sc=False, needs_layout_passes=False),
        scratch_types=[
            pltpu.VMEM((KB, 128), I32),
            pltpu.VMEM((KB, 128), I32),
            pltpu.VMEM((NW * KBA, 128), F32),
            pltpu.VMEM((KB, 128), F32),
            pltpu.VMEM((N,), F32),
            pltpu.VMEM((128, CW), F32),
            pltpu.VMEM((128, CW), F32),
            pltpu.VMEM((ZR, CW), F32),
            pltpu.VMEM_SHARED((N, CW), F32),
        ],
    )(xw3, srcm, dstm, wm, s2)


def _k3_body(nchs, CWs, *refs):
    hps = refs[0:4]
    xhs = refs[4:8]
    g1ws = refs[8:12]
    g1bs = refs[12:16]
    repms = refs[16:20]
    rep = refs[20]
    acc = None
    for m in range(4):
        hp = hps[m][...]
        h = hp[0] + hp[1]
        cw = CWs[m]
        mm = None
        for c in range(nchs[m]):
            part = jnp.dot(_leaky(h[c]),
                           g1ws[m][pl.ds(c * cw, cw), :],
                           preferred_element_type=F32)
            mm = part if mm is None else mm + part
        o = _leaky(mm + g1bs[m][...] + xhs[m][...])
        repms[m][...] = o
        acc = o if acc is None else acc + o
    rep[...] = acc * 0.25


def _k3(hparts, xhats, g1ws, g1bs, nchs, CWs, Lps):
    rb = 400
    in_specs = []
    for m in range(4):
        in_specs.append(pl.BlockSpec((NC, nchs[m], rb, CWs[m]),
                                     lambda i: (0, 0, i, 0)))
    for m in range(4):
        in_specs.append(pl.BlockSpec((rb, DX), lambda i: (i, 0)))
    for m in range(4):
        in_specs.append(pl.BlockSpec((Lps[m], DX), lambda i: (0, 0)))
    for m in range(4):
        in_specs.append(pl.BlockSpec((1, DX), lambda i: (0, 0)))
    return pl.pallas_call(
        functools.partial(_k3_body, nchs, CWs),
        grid=(25,),
        in_specs=in_specs,
        out_specs=[pl.BlockSpec((rb, DX), lambda i: (i, 0))] * 5,
        out_shape=[jax.ShapeDtypeStruct((N, DX), F32)] * 5,
    )(*hparts, *xhats, *g1ws, *g1bs)


def _sct_body(rep, rv, rt, rp, rtr, unm, pim, nim,
              user_o, pos_o, neg_o, kpart,
              idxb, rows, kp_vm):
    cid = lax.axis_index("c")
    sid = lax.axis_index("s")
    wid = sid * NC + cid
    bpw = B // NW

    def gather_out(idx_hbm, out_hbm):
        pltpu.sync_copy(idx_hbm, idxb)
        pltpu.sync_copy(rep.at[idxb.at[wid]], rows)
        pltpu.sync_copy(rows, out_hbm.at[pl.ds(wid * bpw, bpw)])

    gather_out(unm, user_o)
    gather_out(nim, neg_o)
    gather_out(pim, pos_o)

    for m, tab in enumerate((rv, rt, rp, rtr)):
        pltpu.sync_copy(tab.at[idxb.at[wid]], rows)
        for j in range(DX // 16):
            def acc_body(e, a):
                return a + rows[e, pl.ds(j * 16, 16)]
            a = lax.fori_loop(0, bpw, acc_body, jnp.zeros((16,), F32),
                              unroll=8)
            kp_vm[m, pl.ds(j * 16, 16)] = a
    for m in range(4, 8):
        for j in range(DX // 16):
            kp_vm[m, pl.ds(j * 16, 16)] = jnp.zeros((16,), F32)
    pltpu.sync_copy(kp_vm, kpart.at[wid])


def _sct(rep, repms, unm, pim, nim):
    mesh = plsc.VectorSubcoreMesh(core_axis_name="c", subcore_axis_name="s", num_cores=NC, num_subcores=NS)
    bpw = B // NW
    return pl.kernel(
        _sct_body,
        out_type=[
            jax.ShapeDtypeStruct((B, DX), F32),
            jax.ShapeDtypeStruct((B, DX), F32),
            jax.ShapeDtypeStruct((B, DX), F32),
            jax.ShapeDtypeStruct((NW, 8, DX), F32),
        ],
        mesh=mesh,
        compiler_params=pltpu.CompilerParams(use_tc_tiling_on_sc=False, needs_layout_passes=False),
        scratch_types=[
            pltpu.VMEM((NW, bpw), I32),
            pltpu.VMEM((bpw, DX), F32),
            pltpu.VMEM((8, DX), F32),
        ],
    )(rep, *repms, unm, pim, nim)


def _ktail_body(u_ref, p_ref, n_ref, kp_ref, qt_ref, kt_ref, vt_ref,
                w1a_ref, w1b_ref, b1_ref, w2_ref, b2_ref,
                pos_ref, neg_ref, price_ref):
    K8 = jnp.sum(kp_ref[...], axis=0) * (1.0 / B)
    Kp = jnp.dot(K8, kt_ref[...], preferred_element_type=F32)
    Vp = jnp.dot(K8, vt_ref[...], preferred_element_type=F32)
    Q = jnp.dot(u_ref[...], qt_ref[...], preferred_element_type=F32)
    logits = lax.dot_general(Q, Kp, (((1,), (1,)), ((), ())),
                             preferred_element_type=F32) * (1.0 / 8.0)
    col = lax.broadcasted_iota(I32, (B, 8), 1)
    logits = jnp.where(col < 4, logits, -1e30)
    mx = jnp.max(logits, axis=1, keepdims=True)
    e = jnp.exp(logits - mx)
    att_w = e / jnp.sum(e, axis=1, keepdims=True)
    att = jnp.dot(att_w, Vp, preferred_element_type=F32)
    pos_t = p_ref[...]
    pos_ref[...] = jnp.sum(att * pos_t, axis=1, keepdims=True)
    neg_ref[...] = jnp.sum(att * n_ref[...], axis=1, keepdims=True)
    hid = _leaky(jnp.dot(att, w1a_ref[...], preferred_element_type=F32)
                 + jnp.dot(pos_t, w1b_ref[...], preferred_element_type=F32)
                 + b1_ref[...])
    pr = jnp.dot(hid, w2_ref[...], preferred_element_type=F32) + b2_ref[...]
    price_ref[...] = 1.0 / (1.0 + jnp.exp(-pr[:, 0:1]))


def _ktail(user_t, pos_t, neg_t, kpart, qT, kT, vT, w1a, w1b, b1, w2p, b2p):
    full = lambda *s: pl.BlockSpec(s, lambda: tuple(0 for _ in s))
    return pl.pallas_call(
        _ktail_body,
        in_specs=[
            full(B, DX), full(B, DX), full(B, DX), full(NW, 8, DX),
            full(DX, DX), full(DX, DX), full(DX, DX),
            full(DX, DX), full(DX, DX), full(1, DX),
            full(DX, 128), full(1, 128),
        ],
        out_specs=[full(B, 1), full(B, 1), full(B, 1)],
        out_shape=[
            jax.ShapeDtypeStruct((B, 1), F32),
            jax.ShapeDtypeStruct((B, 1), F32),
            jax.ShapeDtypeStruct((B, 1), F32),
        ],
    )(user_t, pos_t, neg_t, kpart, qT, kT, vT, w1a, w1b, b1, w2p, b2p)


def kernel(v_feat, t_feat, p_feat, tr_feat, user_feat, edge_index,
           v_mlp_w, v_mlp_b, v_umlp_w, v_umlp_b, v_conv_w, v_lin1_w,
           v_lin1_b, v_g1_w, v_g1_b,
           t_mlp_w, t_mlp_b, t_umlp_w, t_umlp_b, t_conv_w, t_lin1_w,
           t_lin1_b, t_g1_w, t_g1_b,
           p_mlp_w, p_mlp_b, p_umlp_w, p_umlp_b, p_conv_w, p_lin1_w,
           p_lin1_b, p_g1_w, p_g1_b,
           tr_mlp_w, tr_mlp_b, tr_umlp_w, tr_umlp_b, tr_conv_w, tr_lin1_w,
           tr_lin1_b, tr_g1_w, tr_g1_b,
           id_emb, q_w, k_w, v_w, price_w1, price_b1, price_w2, price_b2,
           user_nodes, pos_items, neg_items):
    mods = {
        "v": (v_feat, v_mlp_w, v_mlp_b, v_umlp_w, v_umlp_b, v_conv_w,
              v_lin1_w, v_lin1_b, v_g1_w, v_g1_b, 128, 1024, 1024, 128),
        "t": (t_feat, t_mlp_w, t_mlp_b, t_umlp_w, t_umlp_b, t_conv_w,
              t_lin1_w, t_lin1_b, t_g1_w, t_g1_b, 128, 1500, 1536, 128),
        "p": (p_feat, p_mlp_w, p_mlp_b, p_umlp_w, p_umlp_b, p_conv_w,
              p_lin1_w, p_lin1_b, p_g1_w, p_g1_b, 32, 64, 64, 64),
        "tr": (tr_feat, tr_mlp_w, tr_mlp_b, tr_umlp_w, tr_umlp_b, tr_conv_w,
               tr_lin1_w, tr_lin1_b, tr_g1_w, tr_g1_b, 32, 64, 64, 64),
    }

    src = edge_index[0]
    dst = edge_index[1]
    padi = jnp.zeros((EP - E,), I32)
    srcm = jnp.concatenate([src, padi]).reshape(NW * KB, 128)
    dstm = jnp.concatenate([dst, padi]).reshape(NW * KB, 128)
    padh = jnp.zeros((EPA - EH,), I32)
    srch = jnp.concatenate([src[:EH], padh]).reshape(NW * KBA, 128)
    dsth = jnp.concatenate([dst[:EH], padh]).reshape(NW * KBA, 128)

    hparts, xhats, g1ws, g1bs, nchs, CWs, Lps = [], [], [], [], [], [], []
    sparts = []
    scb_args = []
    for name, (feat, mw, mb, uw, ub, cw, lw, lb, g1w, g1b,
               Fm, L, Lp, CW) in mods.items():
        nch = Lp // 64
        x, xhat = _k1a(feat, user_feat, mw, mb.reshape(1, L), uw,
                       ub.reshape(1, L), lw, lb.reshape(1, DX), id_emb, L, Fm)
        convp = cw if Lp == L else jnp.pad(cw, ((0, 0), (0, Lp - L)))
        xw3, xw2 = _k1b(x, convp, L, Lp, CW)
        CW = 64
        spart, wout = _sca(xw2, srch, dsth, Lp)
        sparts.append(spart.reshape(NW, N))
        scb_args.append((xw3, wout, nch, CW))
        xhats.append(xhat)
        g1ws.append(g1w if Lp == L else jnp.pad(g1w, ((0, Lp - L), (0, 0))))
        g1bs.append(g1b.reshape(1, DX))
        nchs.append(nch)
        CWs.append(CW)
        Lps.append(Lp)

    s2s = _k2s(sparts)
    for (xw3, wout, nch, CW), s2 in zip(scb_args, s2s):
        hparts.append(_scb(xw3, srcm, dstm, wout,
                           s2, nch, CW))

    repm_v, repm_t, repm_p, repm_tr, rep = _k3(
        hparts, xhats, g1ws, g1bs, nchs, CWs, Lps)

    bpw = B // NW
    unm = user_nodes.reshape(NW, bpw)
    pim = pos_items.reshape(NW, bpw)
    nim = neg_items.reshape(NW, bpw)
    user_t, pos_t, neg_t, kpart = _sct(
        rep, (repm_v, repm_t, repm_p, repm_tr), unm, pim, nim)

    w2p = jnp.pad(price_w2, ((0, 0), (0, 127)))
    b2p = jnp.pad(price_b2.reshape(1, 1), ((0, 0), (0, 127)))
    pos_s, neg_s, price = _ktail(
        user_t, pos_t, neg_t, kpart, q_w.T, k_w.T, v_w.T,
        price_w1[:DX], price_w1[DX:], price_b1.reshape(1, DX), w2p, b2p)

    return (pos_s.reshape(B), neg_s.reshape(B), rep, price)


# speedup vs baseline: 1.4461x; 1.0767x over previous
import functools
import jax
import jax.numpy as jnp
from jax import lax
from jax.experimental import pallas as pl
from jax.experimental.pallas import tpu as pltpu
from jax.experimental.pallas import tpu_sc as plsc

N = 10000
NUM_ITEM = 8000
NUM_USER = 2000
E = 60000
B = 1024
DX = 64

NC = 2
NS = 16
NW = NC * NS
EW = 2048
EP = NW * EW
KB = 16
ECA = 32
NKA = EW // ECA
ZR = 80
NZC = N // ZR
EH = E // 2
EWA = 1024
EPA = NW * EWA
KBA = EWA // 128
NKA2 = EWA // ECA

F32 = jnp.float32
I32 = jnp.int32


def _leaky(x):
    return jnp.where(x >= 0, x, 0.01 * x)


def _k1a_body(feat_ref, user_ref, mw_ref, mb_ref, uw_ref, ub_ref,
              lw_ref, lb_ref, id_ref, x_ref, xhat_ref):
    pid = pl.program_id(0)

    @pl.when(pid < 8)
    def _():
        x_ref[...] = jnp.tanh(
            jnp.dot(feat_ref[...], mw_ref[...], preferred_element_type=F32)
            + mb_ref[...])

    @pl.when(pid >= 8)
    def _():
        x_ref[...] = jnp.tanh(
            jnp.dot(user_ref[...], uw_ref[...], preferred_element_type=F32)
            + ub_ref[...])

    r = x_ref[...]
    nrm = jnp.sqrt(jnp.sum(r * r, axis=1, keepdims=True))
    xl = r / jnp.maximum(nrm, 1e-12)
    x_ref[...] = xl
    xhat_ref[...] = _leaky(
        jnp.dot(xl, lw_ref[...], preferred_element_type=F32)
        + lb_ref[...]) + id_ref[...]


def _k1a(feat, user, mw, mb, uw, ub, lw, lb, id_emb, L, Fm):
    rb = 1000
    return pl.pallas_call(
        _k1a_body,
        grid=(10,),
        in_specs=[
            pl.BlockSpec((rb, Fm), lambda i: (jnp.minimum(i, 7), 0)),
            pl.BlockSpec((rb, 128), lambda i: (jnp.maximum(i - 8, 0), 0)),
            pl.BlockSpec((Fm, L), lambda i: (0, 0)),
            pl.BlockSpec((1, L), lambda i: (0, 0)),
            pl.BlockSpec((128, L), lambda i: (0, 0)),
            pl.BlockSpec((1, L), lambda i: (0, 0)),
            pl.BlockSpec((L, DX), lambda i: (0, 0)),
            pl.BlockSpec((1, DX), lambda i: (0, 0)),
            pl.BlockSpec((rb, DX), lambda i: (i, 0)),
        ],
        out_specs=[
            pl.BlockSpec((rb, L), lambda i: (i, 0)),
            pl.BlockSpec((rb, DX), lambda i: (i, 0)),
        ],
        out_shape=[
            jax.ShapeDtypeStruct((N, L), F32),
            jax.ShapeDtypeStruct((N, DX), F32),
        ],
    )(feat, user, mw, mb, uw, ub, lw, lb, id_emb)


def _k1b_body(nsub, x_ref, cw_ref, xw3_ref, xw2_ref):
    t = jnp.dot(x_ref[...], cw_ref[...], preferred_element_type=F32)
    for u in range(nsub):
        xw3_ref[u] = t[:, u * 64:(u + 1) * 64]
    xw2_ref[...] = t


def _k1b(x, convp, L, Lp, CWT):
    rb = 1000
    nch = Lp // CWT
    nsub = CWT // 64
    return pl.pallas_call(
        functools.partial(_k1b_body, nsub),
        grid=(10, nch),
        in_specs=[
            pl.BlockSpec((rb, L), lambda i, j: (i, 0)),
            pl.BlockSpec((L, CWT), lambda i, j: (0, j)),
        ],
        out_specs=[
            pl.BlockSpec((nsub, rb, 64), lambda i, j: (j, i, 0)),
            pl.BlockSpec((rb, CWT), lambda i, j: (i, j)),
        ],
        out_shape=[
            jax.ShapeDtypeStruct((Lp // 64, N, 64), F32),
            jax.ShapeDtypeStruct((N, Lp), F32),
        ],
    )(x, convp)


def _sca_body(Lp, xw2, srcm, dstm, spart, wout,
              idx_s, idx_d, rows_s, rows_d, wbuf, s_loc, gsem):
    cid = lax.axis_index("c")
    sid = lax.axis_index("s")
    wid = sid * NC + cid
    iota = lax.iota(I32, 16)

    pltpu.sync_copy(srcm.at[pl.ds(wid * KBA, KBA)], idx_s)
    pltpu.sync_copy(dstm.at[pl.ds(wid * KBA, KBA)], idx_d)

    def zero_body(i, _):
        s_loc[pl.ds(i * 16, 16)] = jnp.zeros((16,), F32)
        return 0
    lax.fori_loop(0, N // 16, zero_body, 0)

    def k_body(k, _):
        kb = k // 4
        off = (k % 4) * ECA
        cp1 = pltpu.async_copy(xw2.at[idx_s.at[kb, pl.ds(off, ECA)]],
                               rows_s, gsem)
        cp2 = pltpu.async_copy(xw2.at[idx_d.at[kb, pl.ds(off, ECA)]],
                               rows_d, gsem)
        cp1.wait()
        cp2.wait()

        def col_body(j, acc):
            a0, a1 = acc
            jv = jnp.full((16,), j, I32)
            s0 = plsc.load_gather(rows_s, [iota, jv])
            d0 = plsc.load_gather(rows_d, [iota, jv])
            s1 = plsc.load_gather(rows_s, [iota + 16, jv])
            d1 = plsc.load_gather(rows_d, [iota + 16, jv])
            return (a0 + s0 * d0, a1 + s1 * d1)

        z = jnp.zeros((16,), F32)
        a0, a1 = lax.fori_loop(0, Lp, col_body, (z, z), unroll=8)

        for g, inner in ((0, a0), (1, a1)):
            eid = wid * EWA + k * ECA + g * 16 + iota
            mask = eid < EH
            ev = jnp.where(mask, jnp.exp(inner), 0.0)
            gate = 1.0 / (1.0 + jnp.exp(-inner))
            w = jnp.where(mask, ev * gate, 0.0)
            wbuf[kb, pl.ds(off + g * 16, 16)] = w
            dst16 = idx_d[kb, pl.ds(off + g * 16, 16)]
            src16 = idx_s[kb, pl.ds(off + g * 16, 16)]
            plsc.addupdate_scatter(s_loc, [dst16], ev)
            plsc.addupdate_scatter(s_loc, [src16], ev)
        return 0

    lax.fori_loop(0, NKA2, k_body, 0)
    pltpu.sync_copy(s_loc, spart.at[pl.ds(wid * N, N)])
    pltpu.sync_copy(wbuf, wout.at[pl.ds(wid * KBA, KBA)])


def _sca(xw2, srcm, dstm, Lp):
    mesh = plsc.VectorSubcoreMesh(core_axis_name="c", subcore_axis_name="s", num_cores=NC, num_subcores=NS)
    return pl.kernel(
        functools.partial(_sca_body, Lp),
        out_type=[
            jax.ShapeDtypeStruct((NW * N,), F32),
            jax.ShapeDtypeStruct((NW * KBA, 128), F32),
        ],
        mesh=mesh,
        compiler_params=pltpu.CompilerParams(use_tc_tiling_on_sc=False, needs_layout_passes=False),
        scratch_types=[
            pltpu.VMEM((KBA, 128), I32),
            pltpu.VMEM((KBA, 128), I32),
            pltpu.VMEM((ECA, Lp), F32),
            pltpu.VMEM((ECA, Lp), F32),
            pltpu.VMEM((KBA, 128), F32),
            pltpu.VMEM((N,), F32),
            pltpu.SemaphoreType.DMA,
        ],
    )(xw2, srcm, dstm)


def _k2s_body(*refs):
    ins = refs[:4]
    outs = refs[4:]
    for i_ref, o_ref in zip(ins, outs):
        o_ref[...] = jnp.sum(i_ref[...], axis=0, keepdims=True)


def _k2s(sparts):
    return pl.pallas_call(
        _k2s_body,
        in_specs=[pl.BlockSpec((NW, N), lambda: (0, 0))] * 4,
        out_specs=[pl.BlockSpec((1, N), lambda: (0, 0))] * 4,
        out_shape=[jax.ShapeDtypeStruct((1, N), F32)] * 4,
    )(*sparts)


def _scb_body(nch, CW, xw3, srcm, dstm, wm, sm, hpart,
              idx_s, idx_d, wvm, alpha, s_vm, rows, scaled, zbuf, hsh, gsem):
    cid = lax.axis_index("c")
    sid = lax.axis_index("s")
    wid = sid * NC + cid

    iota = lax.iota(I32, 16)
    pltpu.sync_copy(srcm.at[pl.ds(wid * KB, KB)], idx_s)
    pltpu.sync_copy(dstm.at[pl.ds(wid * KB, KB)], idx_d)
    start1 = pl.multiple_of(jnp.minimum(wid * EW, EPA - EW), 8)
    start2 = pl.multiple_of(jnp.clip(wid * EW - EH, 0, EPA - EW), 8)
    pltpu.sync_copy(wm.at[pl.ds(start1, EW)], wvm.at[pl.ds(0, EW)])
    pltpu.sync_copy(wm.at[pl.ds(start2, EW)], wvm.at[pl.ds(EW, EW)])
    pltpu.sync_copy(sm.at[0], s_vm)

    def zb2_body(i, _):
        r = i // (CW // 16)
        c = (i % (CW // 16)) * 16
        zbuf[r, pl.ds(c, 16)] = jnp.zeros((16,), F32)
        return 0
    lax.fori_loop(0, ZR * (CW // 16), zb2_body, 0, unroll=8)

    def al_body(i, _):
        kb = i // 8
        off = (i % 8) * 16
        eid = wid * EW + i * 16 + iota
        mask = eid < E
        woff = jnp.where(eid < EH, eid - start1,
                         jnp.clip(eid - EH - start2, 0, EW - 1) + EW)
        woff = jnp.where(mask, woff, 0)
        dst16 = idx_d[kb, pl.ds(off, 16)]
        sv = plsc.load_gather(s_vm, [dst16])
        w16 = plsc.load_gather(wvm, [woff])
        a16 = jnp.where(mask, w16 / (sv + 1e-16), 0.0)
        alpha[kb, pl.ds(off, 16)] = a16
        return 0
    lax.fori_loop(0, KB * 8, al_body, 0)

    def chunk_body(c, _):
        def z_body(j, _):
            i = sid + j * NS

            @pl.when(i < NZC)
            def _():
                pltpu.sync_copy(zbuf, hsh.at[pl.ds(i * ZR, ZR)])
            return 0
        lax.fori_loop(0, (NZC + NS - 1) // NS, z_body, 0)
        plsc.subcore_barrier()

        def k4_body(kb4, _):
            cps = [
                pltpu.async_copy(xw3.at[c].at[idx_s.at[kb4 * 4 + u]],
                                 rows.at[pl.ds(u * 128, 128)], gsem)
                for u in range(4)
            ]
            for cp in cps:
                cp.wait()
            for u in range(4):
                kbg = kb4 * 4 + u

                def e_body(e, _, u=u, kbg=kbg):
                    sp = plsc.load_gather(
                        alpha, [jnp.full((16,), kbg, I32),
                                jnp.full((16,), e, I32)])
                    for j in range(CW // 16):
                        scaled[e, pl.ds(j * 16, 16)] = (
                            rows[u * 128 + e, pl.ds(j * 16, 16)] * sp)
                    return 0
                lax.fori_loop(0, 128, e_body, 0, unroll=4)
                pltpu.sync_copy(scaled, hsh.at[idx_d.at[kbg]], add=True)
            return 0
        lax.fori_loop(0, KB // 4, k4_body, 0)
        plsc.subcore_barrier()

        def o_body(j, _):
            i = sid + j * NS

            @pl.when(i < NZC)
            def _():
                pltpu.sync_copy(hsh.at[pl.ds(i * ZR, ZR)],
                                hpart.at[cid, c, pl.ds(i * ZR, ZR)])
            return 0
        lax.fori_loop(0, (NZC + NS - 1) // NS, o_body, 0)
        plsc.subcore_barrier()
        return 0

    lax.fori_loop(0, nch, chunk_body, 0)


def _scb(xw3, srcm, dstm, wm, s2, nch, CW):
    mesh = plsc.VectorSubcoreMesh(core_axis_name="c", subcore_axis_name="s", num_cores=NC, num_subcores=NS)
    return pl.kernel(
        functools.partial(_scb_body, nch, CW),
        out_type=jax.ShapeDtypeStruct((NC, nch, N, CW), F32),
        mesh=mesh,
        compiler_params=pltpu.CompilerParams(use_tc_tiling_on_sc=False, needs_layout_passes=False),
        scratch_types=[
            pltpu.VMEM((KB, 128), I32),
            pltpu.VMEM((KB, 128), I32),
            pltpu.VMEM((2 * EW,), F32),
            pltpu.VMEM((KB, 128), F32),
            pltpu.VMEM((N,), F32),
            pltpu.VMEM((512, CW), F32),
            pltpu.VMEM((128, CW), F32),
            pltpu.VMEM((ZR, CW), F32),
            pltpu.VMEM_SHARED((N, CW), F32),
            pltpu.SemaphoreType.DMA,
        ],
    )(xw3, srcm, dstm, wm, s2)


def _k3_body(nchs, CWs, *refs):
    hps = refs[0:4]
    xhs = refs[4:8]
    g1ws = refs[8:12]
    g1bs = refs[12:16]
    repms = refs[16:20]
    rep = refs[20]
    acc = None
    for m in range(4):
        hp = hps[m][...]
        h = hp[0] + hp[1]
        cw = CWs[m]
        mm = None
        for c in range(nchs[m]):
            part = jnp.dot(_leaky(h[c]),
                           g1ws[m][pl.ds(c * cw, cw), :],
                           preferred_element_type=F32)
            mm = part if mm is None else mm + part
        o = _leaky(mm + g1bs[m][...] + xhs[m][...])
        repms[m][...] = o
        acc = o if acc is None else acc + o
    rep[...] = acc * 0.25


def _k3(hparts, xhats, g1ws, g1bs, nchs, CWs, Lps):
    rb = 400
    in_specs = []
    for m in range(4):
        in_specs.append(pl.BlockSpec((NC, nchs[m], rb, CWs[m]),
                                     lambda i: (0, 0, i, 0)))
    for m in range(4):
        in_specs.append(pl.BlockSpec((rb, DX), lambda i: (i, 0)))
    for m in range(4):
        in_specs.append(pl.BlockSpec((Lps[m], DX), lambda i: (0, 0)))
    for m in range(4):
        in_specs.append(pl.BlockSpec((1, DX), lambda i: (0, 0)))
    return pl.pallas_call(
        functools.partial(_k3_body, nchs, CWs),
        grid=(25,),
        in_specs=in_specs,
        out_specs=[pl.BlockSpec((rb, DX), lambda i: (i, 0))] * 5,
        out_shape=[jax.ShapeDtypeStruct((N, DX), F32)] * 5,
    )(*hparts, *xhats, *g1ws, *g1bs)


def _sct_body(rep, rv, rt, rp, rtr, unm, pim, nim,
              user_o, pos_o, neg_o, kpart,
              idxb, rows, kp_vm):
    cid = lax.axis_index("c")
    sid = lax.axis_index("s")
    wid = sid * NC + cid
    bpw = B // NW

    def gather_out(idx_hbm, out_hbm):
        pltpu.sync_copy(idx_hbm, idxb)
        pltpu.sync_copy(rep.at[idxb.at[wid]], rows)
        pltpu.sync_copy(rows, out_hbm.at[pl.ds(wid * bpw, bpw)])

    gather_out(unm, user_o)
    gather_out(nim, neg_o)
    gather_out(pim, pos_o)

    for m, tab in enumerate((rv, rt, rp, rtr)):
        pltpu.sync_copy(tab.at[idxb.at[wid]], rows)
        for j in range(DX // 16):
            def acc_body(e, a):
                return a + rows[e, pl.ds(j * 16, 16)]
            a = lax.fori_loop(0, bpw, acc_body, jnp.zeros((16,), F32),
                              unroll=8)
            kp_vm[m, pl.ds(j * 16, 16)] = a
    for m in range(4, 8):
        for j in range(DX // 16):
            kp_vm[m, pl.ds(j * 16, 16)] = jnp.zeros((16,), F32)
    pltpu.sync_copy(kp_vm, kpart.at[wid])


def _sct(rep, repms, unm, pim, nim):
    mesh = plsc.VectorSubcoreMesh(core_axis_name="c", subcore_axis_name="s", num_cores=NC, num_subcores=NS)
    bpw = B // NW
    return pl.kernel(
        _sct_body,
        out_type=[
            jax.ShapeDtypeStruct((B, DX), F32),
            jax.ShapeDtypeStruct((B, DX), F32),
            jax.ShapeDtypeStruct((B, DX), F32),
            jax.ShapeDtypeStruct((NW, 8, DX), F32),
        ],
        mesh=mesh,
        compiler_params=pltpu.CompilerParams(use_tc_tiling_on_sc=False, needs_layout_passes=False),
        scratch_types=[
            pltpu.VMEM((NW, bpw), I32),
            pltpu.VMEM((bpw, DX), F32),
            pltpu.VMEM((8, DX), F32),
        ],
    )(rep, *repms, unm, pim, nim)


def _ktail_body(u_ref, p_ref, n_ref, kp_ref, qt_ref, kt_ref, vt_ref,
                w1a_ref, w1b_ref, b1_ref, w2_ref, b2_ref,
                pos_ref, neg_ref, price_ref):
    K8 = jnp.sum(kp_ref[...], axis=0) * (1.0 / B)
    Kp = jnp.dot(K8, kt_ref[...], preferred_element_type=F32)
    Vp = jnp.dot(K8, vt_ref[...], preferred_element_type=F32)
    Q = jnp.dot(u_ref[...], qt_ref[...], preferred_element_type=F32)
    logits = lax.dot_general(Q, Kp, (((1,), (1,)), ((), ())),
                             preferred_element_type=F32) * (1.0 / 8.0)
    col = lax.broadcasted_iota(I32, (B, 8), 1)
    logits = jnp.where(col < 4, logits, -1e30)
    mx = jnp.max(logits, axis=1, keepdims=True)
    e = jnp.exp(logits - mx)
    att_w = e / jnp.sum(e, axis=1, keepdims=True)
    att = jnp.dot(att_w, Vp, preferred_element_type=F32)
    pos_t = p_ref[...]
    pos_ref[...] = jnp.sum(att * pos_t, axis=1, keepdims=True)
    neg_ref[...] = jnp.sum(att * n_ref[...], axis=1, keepdims=True)
    hid = _leaky(jnp.dot(att, w1a_ref[...], preferred_element_type=F32)
                 + jnp.dot(pos_t, w1b_ref[...], preferred_element_type=F32)
                 + b1_ref[...])
    pr = jnp.dot(hid, w2_ref[...], preferred_element_type=F32) + b2_ref[...]
    price_ref[...] = 1.0 / (1.0 + jnp.exp(-pr[:, 0:1]))


def _ktail(user_t, pos_t, neg_t, kpart, qT, kT, vT, w1a, w1b, b1, w2p, b2p):
    full = lambda *s: pl.BlockSpec(s, lambda: tuple(0 for _ in s))
    return pl.pallas_call(
        _ktail_body,
        in_specs=[
            full(B, DX), full(B, DX), full(B, DX), full(NW, 8, DX),
            full(DX, DX), full(DX, DX), full(DX, DX),
            full(DX, DX), full(DX, DX), full(1, DX),
            full(DX, 128), full(1, 128),
        ],
        out_specs=[full(B, 1), full(B, 1), full(B, 1)],
        out_shape=[
            jax.ShapeDtypeStruct((B, 1), F32),
            jax.ShapeDtypeStruct((B, 1), F32),
            jax.ShapeDtypeStruct((B, 1), F32),
        ],
    )(user_t, pos_t, neg_t, kpart, qT, kT, vT, w1a, w1b, b1, w2p, b2p)


def kernel(v_feat, t_feat, p_feat, tr_feat, user_feat, edge_index,
           v_mlp_w, v_mlp_b, v_umlp_w, v_umlp_b, v_conv_w, v_lin1_w,
           v_lin1_b, v_g1_w, v_g1_b,
           t_mlp_w, t_mlp_b, t_umlp_w, t_umlp_b, t_conv_w, t_lin1_w,
           t_lin1_b, t_g1_w, t_g1_b,
           p_mlp_w, p_mlp_b, p_umlp_w, p_umlp_b, p_conv_w, p_lin1_w,
           p_lin1_b, p_g1_w, p_g1_b,
           tr_mlp_w, tr_mlp_b, tr_umlp_w, tr_umlp_b, tr_conv_w, tr_lin1_w,
           tr_lin1_b, tr_g1_w, tr_g1_b,
           id_emb, q_w, k_w, v_w, price_w1, price_b1, price_w2, price_b2,
           user_nodes, pos_items, neg_items):
    mods = {
        "v": (v_feat, v_mlp_w, v_mlp_b, v_umlp_w, v_umlp_b, v_conv_w,
              v_lin1_w, v_lin1_b, v_g1_w, v_g1_b, 128, 1024, 1024, 128),
        "t": (t_feat, t_mlp_w, t_mlp_b, t_umlp_w, t_umlp_b, t_conv_w,
              t_lin1_w, t_lin1_b, t_g1_w, t_g1_b, 128, 1500, 1536, 128),
        "p": (p_feat, p_mlp_w, p_mlp_b, p_umlp_w, p_umlp_b, p_conv_w,
              p_lin1_w, p_lin1_b, p_g1_w, p_g1_b, 32, 64, 64, 64),
        "tr": (tr_feat, tr_mlp_w, tr_mlp_b, tr_umlp_w, tr_umlp_b, tr_conv_w,
               tr_lin1_w, tr_lin1_b, tr_g1_w, tr_g1_b, 32, 64, 64, 64),
    }

    src = edge_index[0]
    dst = edge_index[1]
    padi = jnp.zeros((EP - E,), I32)
    srcm = jnp.concatenate([src, padi]).reshape(NW * KB, 128)
    dstm = jnp.concatenate([dst, padi]).reshape(NW * KB, 128)
    padh = jnp.zeros((EPA - EH,), I32)
    srch = jnp.concatenate([src[:EH], padh]).reshape(NW * KBA, 128)
    dsth = jnp.concatenate([dst[:EH], padh]).reshape(NW * KBA, 128)

    hparts, xhats, g1ws, g1bs, nchs, CWs, Lps = [], [], [], [], [], [], []
    sparts = []
    scb_args = []
    for name, (feat, mw, mb, uw, ub, cw, lw, lb, g1w, g1b,
               Fm, L, Lp, CW) in mods.items():
        nch = Lp // 64
        x, xhat = _k1a(feat, user_feat, mw, mb.reshape(1, L), uw,
                       ub.reshape(1, L), lw, lb.reshape(1, DX), id_emb, L, Fm)
        convp = cw if Lp == L else jnp.pad(cw, ((0, 0), (0, Lp - L)))
        xw3, xw2 = _k1b(x, convp, L, Lp, CW)
        CW = 64
        spart, wout = _sca(xw2, srch, dsth, Lp)
        sparts.append(spart.reshape(NW, N))
        scb_args.append((xw3, wout, nch, CW))
        xhats.append(xhat)
        g1ws.append(g1w if Lp == L else jnp.pad(g1w, ((0, Lp - L), (0, 0))))
        g1bs.append(g1b.reshape(1, DX))
        nchs.append(nch)
        CWs.append(CW)
        Lps.append(Lp)

    s2s = _k2s(sparts)
    for (xw3, wout, nch, CW), s2 in zip(scb_args, s2s):
        hparts.append(_scb(xw3, srcm, dstm, wout.reshape(EPA),
                           s2, nch, CW))

    repm_v, repm_t, repm_p, repm_tr, rep = _k3(
        hparts, xhats, g1ws, g1bs, nchs, CWs, Lps)

    bpw = B // NW
    unm = user_nodes.reshape(NW, bpw)
    pim = pos_items.reshape(NW, bpw)
    nim = neg_items.reshape(NW, bpw)
    user_t, pos_t, neg_t, kpart = _sct(
        rep, (repm_v, repm_t, repm_p, repm_tr), unm, pim, nim)

    w2p = jnp.pad(price_w2, ((0, 0), (0, 127)))
    b2p = jnp.pad(price_b2.reshape(1, 1), ((0, 0), (0, 127)))
    pos_s, neg_s, price = _ktail(
        user_t, pos_t, neg_t, kpart, q_w.T, k_w.T, v_w.T,
        price_w1[:DX], price_w1[DX:], price_b1.reshape(1, DX), w2p, b2p)

    return (pos_s.reshape(B), neg_s.reshape(B), rep, price)


# speedup vs baseline: 1.4601x; 1.0097x over previous
import functools
import jax
import jax.numpy as jnp
from jax import lax
from jax.experimental import pallas as pl
from jax.experimental.pallas import tpu as pltpu
from jax.experimental.pallas import tpu_sc as plsc

N = 10000
NUM_ITEM = 8000
NUM_USER = 2000
E = 60000
B = 1024
DX = 64

NC = 2
NS = 16
NW = NC * NS
EW = 2048
EP = NW * EW
KB = 16
ECA = 32
NKA = EW // ECA
ZR = 400
NZC = N // ZR
EH = E // 2
EWA = 1024
EPA = NW * EWA
KBA = EWA // 128
NKA2 = EWA // ECA

F32 = jnp.float32
I32 = jnp.int32


def _leaky(x):
    return jnp.where(x >= 0, x, 0.01 * x)


def _k1a_body(feat_ref, user_ref, mw_ref, mb_ref, uw_ref, ub_ref,
              lw_ref, lb_ref, id_ref, x_ref, xhat_ref):
    pid = pl.program_id(0)

    @pl.when(pid < 8)
    def _():
        x_ref[...] = jnp.tanh(
            jnp.dot(feat_ref[...], mw_ref[...], preferred_element_type=F32)
            + mb_ref[...])

    @pl.when(pid >= 8)
    def _():
        x_ref[...] = jnp.tanh(
            jnp.dot(user_ref[...], uw_ref[...], preferred_element_type=F32)
            + ub_ref[...])

    r = x_ref[...]
    nrm = jnp.sqrt(jnp.sum(r * r, axis=1, keepdims=True))
    xl = r / jnp.maximum(nrm, 1e-12)
    x_ref[...] = xl
    xhat_ref[...] = _leaky(
        jnp.dot(xl, lw_ref[...], preferred_element_type=F32)
        + lb_ref[...]) + id_ref[...]


def _k1a(feat, user, mw, mb, uw, ub, lw, lb, id_emb, L, Fm):
    rb = 1000
    return pl.pallas_call(
        _k1a_body,
        grid=(10,),
        in_specs=[
            pl.BlockSpec((rb, Fm), lambda i: (jnp.minimum(i, 7), 0)),
            pl.BlockSpec((rb, 128), lambda i: (jnp.maximum(i - 8, 0), 0)),
            pl.BlockSpec((Fm, L), lambda i: (0, 0)),
            pl.BlockSpec((1, L), lambda i: (0, 0)),
            pl.BlockSpec((128, L), lambda i: (0, 0)),
            pl.BlockSpec((1, L), lambda i: (0, 0)),
            pl.BlockSpec((L, DX), lambda i: (0, 0)),
            pl.BlockSpec((1, DX), lambda i: (0, 0)),
            pl.BlockSpec((rb, DX), lambda i: (i, 0)),
        ],
        out_specs=[
            pl.BlockSpec((rb, L), lambda i: (i, 0)),
            pl.BlockSpec((rb, DX), lambda i: (i, 0)),
        ],
        out_shape=[
            jax.ShapeDtypeStruct((N, L), F32),
            jax.ShapeDtypeStruct((N, DX), F32),
        ],
    )(feat, user, mw, mb, uw, ub, lw, lb, id_emb)


def _k1b_body(nsub, x_ref, cw_ref, xw3_ref, xw2_ref):
    t = jnp.dot(x_ref[...], cw_ref[...], preferred_element_type=F32)
    for u in range(nsub):
        xw3_ref[u] = t[:, u * 64:(u + 1) * 64]
    xw2_ref[...] = t


def _k1b(x, convp, L, Lp, CWT):
    rb = 1000
    nch = Lp // CWT
    nsub = CWT // 64
    return pl.pallas_call(
        functools.partial(_k1b_body, nsub),
        grid=(10, nch),
        in_specs=[
            pl.BlockSpec((rb, L), lambda i, j: (i, 0)),
            pl.BlockSpec((L, CWT), lambda i, j: (0, j)),
        ],
        out_specs=[
            pl.BlockSpec((nsub, rb, 64), lambda i, j: (j, i, 0)),
            pl.BlockSpec((rb, CWT), lambda i, j: (i, j)),
        ],
        out_shape=[
            jax.ShapeDtypeStruct((Lp // 64, N, 64), F32),
            jax.ShapeDtypeStruct((N, Lp), F32),
        ],
    )(x, convp)


def _sca_body(Lp, xw2, srcm, dstm, spart, wout,
              idx_s, idx_d, rows_s, rows_d, wbuf, s_loc, gsem):
    cid = lax.axis_index("c")
    sid = lax.axis_index("s")
    wid = sid * NC + cid
    iota = lax.iota(I32, 16)

    pltpu.sync_copy(srcm.at[pl.ds(wid * KBA, KBA)], idx_s)
    pltpu.sync_copy(dstm.at[pl.ds(wid * KBA, KBA)], idx_d)

    def zero_body(i, _):
        s_loc[pl.ds(i * 16, 16)] = jnp.zeros((16,), F32)
        return 0
    lax.fori_loop(0, N // 16, zero_body, 0)

    def k_body(k, _):
        kb = k // 4
        off = (k % 4) * ECA
        cp1 = pltpu.async_copy(xw2.at[idx_s.at[kb, pl.ds(off, ECA)]],
                               rows_s, gsem)
        cp2 = pltpu.async_copy(xw2.at[idx_d.at[kb, pl.ds(off, ECA)]],
                               rows_d, gsem)
        cp1.wait()
        cp2.wait()

        def col_body(j, acc):
            a0, a1 = acc
            jv = jnp.full((16,), j, I32)
            s0 = plsc.load_gather(rows_s, [iota, jv])
            d0 = plsc.load_gather(rows_d, [iota, jv])
            s1 = plsc.load_gather(rows_s, [iota + 16, jv])
            d1 = plsc.load_gather(rows_d, [iota + 16, jv])
            return (a0 + s0 * d0, a1 + s1 * d1)

        z = jnp.zeros((16,), F32)
        a0, a1 = lax.fori_loop(0, Lp, col_body, (z, z), unroll=8)

        for g, inner in ((0, a0), (1, a1)):
            eid = wid * EWA + k * ECA + g * 16 + iota
            mask = eid < EH
            ev = jnp.where(mask, jnp.exp(inner), 0.0)
            gate = 1.0 / (1.0 + jnp.exp(-inner))
            w = jnp.where(mask, ev * gate, 0.0)
            wbuf[kb, pl.ds(off + g * 16, 16)] = w
            dst16 = idx_d[kb, pl.ds(off + g * 16, 16)]
            src16 = idx_s[kb, pl.ds(off + g * 16, 16)]
            plsc.addupdate_scatter(s_loc, [dst16], ev)
            plsc.addupdate_scatter(s_loc, [src16], ev)
        return 0

    lax.fori_loop(0, NKA2, k_body, 0)
    pltpu.sync_copy(s_loc, spart.at[pl.ds(wid * N, N)])
    pltpu.sync_copy(wbuf, wout.at[pl.ds(wid * KBA, KBA)])


def _sca(xw2, srcm, dstm, Lp):
    mesh = plsc.VectorSubcoreMesh(core_axis_name="c", subcore_axis_name="s", num_cores=NC, num_subcores=NS)
    return pl.kernel(
        functools.partial(_sca_body, Lp),
        out_type=[
            jax.ShapeDtypeStruct((NW * N,), F32),
            jax.ShapeDtypeStruct((NW * KBA, 128), F32),
        ],
        mesh=mesh,
        compiler_params=pltpu.CompilerParams(use_tc_tiling_on_sc=False, needs_layout_passes=False),
        scratch_types=[
            pltpu.VMEM((KBA, 128), I32),
            pltpu.VMEM((KBA, 128), I32),
            pltpu.VMEM((ECA, Lp), F32),
            pltpu.VMEM((ECA, Lp), F32),
            pltpu.VMEM((KBA, 128), F32),
            pltpu.VMEM((N,), F32),
            pltpu.SemaphoreType.DMA,
        ],
    )(xw2, srcm, dstm)


def _k2s_body(*refs):
    ins = refs[:4]
    outs = refs[4:]
    for i_ref, o_ref in zip(ins, outs):
        o_ref[...] = jnp.sum(i_ref[...], axis=0, keepdims=True)


def _k2s(sparts):
    return pl.pallas_call(
        _k2s_body,
        in_specs=[pl.BlockSpec((NW, N), lambda: (0, 0))] * 4,
        out_specs=[pl.BlockSpec((1, N), lambda: (0, 0))] * 4,
        out_shape=[jax.ShapeDtypeStruct((1, N), F32)] * 4,
    )(*sparts)


def _scb_body(nch, CW, xw3, srcm, dstm, wm, sm, hpart,
              idx_s, idx_d, wvm, alpha, s_vm, rows, scaled, zbuf, hsh, gsem):
    cid = lax.axis_index("c")
    sid = lax.axis_index("s")
    wid = sid * NC + cid

    iota = lax.iota(I32, 16)
    pltpu.sync_copy(srcm.at[pl.ds(wid * KB, KB)], idx_s)
    pltpu.sync_copy(dstm.at[pl.ds(wid * KB, KB)], idx_d)
    start1 = pl.multiple_of(jnp.minimum(wid * EW, EPA - EW), 8)
    start2 = pl.multiple_of(jnp.clip(wid * EW - EH, 0, EPA - EW), 8)
    pltpu.sync_copy(wm.at[pl.ds(start1, EW)], wvm.at[pl.ds(0, EW)])
    pltpu.sync_copy(wm.at[pl.ds(start2, EW)], wvm.at[pl.ds(EW, EW)])
    pltpu.sync_copy(sm.at[0], s_vm)

    def zb2_body(i, _):
        r = i // (CW // 16)
        c = (i % (CW // 16)) * 16
        zbuf[r, pl.ds(c, 16)] = jnp.zeros((16,), F32)
        return 0
    lax.fori_loop(0, ZR * (CW // 16), zb2_body, 0, unroll=8)

    def al_body(i, _):
        kb = i // 8
        off = (i % 8) * 16
        eid = wid * EW + i * 16 + iota
        mask = eid < E
        woff = jnp.where(eid < EH, eid - start1,
                         jnp.clip(eid - EH - start2, 0, EW - 1) + EW)
        woff = jnp.where(mask, woff, 0)
        dst16 = idx_d[kb, pl.ds(off, 16)]
        sv = plsc.load_gather(s_vm, [dst16])
        w16 = plsc.load_gather(wvm, [woff])
        a16 = jnp.where(mask, w16 / (sv + 1e-16), 0.0)
        alpha[kb, pl.ds(off, 16)] = a16
        return 0
    lax.fori_loop(0, KB * 8, al_body, 0)

    def chunk_body(c, _):
        def z_body(j, _):
            i = sid + j * NS

            @pl.when(i < NZC)
            def _():
                pltpu.sync_copy(zbuf, hsh.at[pl.ds(i * ZR, ZR)])
            return 0
        lax.fori_loop(0, (NZC + NS - 1) // NS, z_body, 0)
        plsc.subcore_barrier()

        def k4_body(kb4, _):
            cps = [
                pltpu.async_copy(xw3.at[c].at[idx_s.at[kb4 * 4 + u]],
                                 rows.at[pl.ds(u * 128, 128)], gsem)
                for u in range(4)
            ]
            for cp in cps:
                cp.wait()
            for u in range(4):
                kbg = kb4 * 4 + u

                def e_body(e, _, u=u, kbg=kbg):
                    sp = plsc.load_gather(
                        alpha, [jnp.full((16,), kbg, I32),
                                jnp.full((16,), e, I32)])
                    for j in range(CW // 16):
                        scaled[e, pl.ds(j * 16, 16)] = (
                            rows[u * 128 + e, pl.ds(j * 16, 16)] * sp)
                    return 0
                lax.fori_loop(0, 128, e_body, 0, unroll=8)
                pltpu.sync_copy(scaled, hsh.at[idx_d.at[kbg]], add=True)
            return 0
        lax.fori_loop(0, KB // 4, k4_body, 0)
        plsc.subcore_barrier()

        def o_body(j, _):
            i = sid + j * NS

            @pl.when(i < NZC)
            def _():
                pltpu.sync_copy(hsh.at[pl.ds(i * ZR, ZR)],
                                hpart.at[cid, c, pl.ds(i * ZR, ZR)])
            return 0
        lax.fori_loop(0, (NZC + NS - 1) // NS, o_body, 0)
        plsc.subcore_barrier()
        return 0

    lax.fori_loop(0, nch, chunk_body, 0)


def _scb(xw3, srcm, dstm, wm, s2, nch, CW):
    mesh = plsc.VectorSubcoreMesh(core_axis_name="c", subcore_axis_name="s", num_cores=NC, num_subcores=NS)
    return pl.kernel(
        functools.partial(_scb_body, nch, CW),
        out_type=jax.ShapeDtypeStruct((NC, nch, N, CW), F32),
        mesh=mesh,
        compiler_params=pltpu.CompilerParams(use_tc_tiling_on_sc=False, needs_layout_passes=False),
        scratch_types=[
            pltpu.VMEM((KB, 128), I32),
            pltpu.VMEM((KB, 128), I32),
            pltpu.VMEM((2 * EW,), F32),
            pltpu.VMEM((KB, 128), F32),
            pltpu.VMEM((N,), F32),
            pltpu.VMEM((512, CW), F32),
            pltpu.VMEM((128, CW), F32),
            pltpu.VMEM((ZR, CW), F32),
            pltpu.VMEM_SHARED((N, CW), F32),
            pltpu.SemaphoreType.DMA,
        ],
    )(xw3, srcm, dstm, wm, s2)


def _k3_body(nchs, CWs, *refs):
    hps = refs[0:4]
    xhs = refs[4:8]
    g1ws = refs[8:12]
    g1bs = refs[12:16]
    repms = refs[16:20]
    rep = refs[20]
    acc = None
    for m in range(4):
        hp = hps[m][...]
        h = hp[0] + hp[1]
        cw = CWs[m]
        mm = None
        for c in range(nchs[m]):
            part = jnp.dot(_leaky(h[c]),
                           g1ws[m][pl.ds(c * cw, cw), :],
                           preferred_element_type=F32)
            mm = part if mm is None else mm + part
        o = _leaky(mm + g1bs[m][...] + xhs[m][...])
        repms[m][...] = o
        acc = o if acc is None else acc + o
    rep[...] = acc * 0.25


def _k3(hparts, xhats, g1ws, g1bs, nchs, CWs, Lps):
    rb = 400
    in_specs = []
    for m in range(4):
        in_specs.append(pl.BlockSpec((NC, nchs[m], rb, CWs[m]),
                                     lambda i: (0, 0, i, 0)))
    for m in range(4):
        in_specs.append(pl.BlockSpec((rb, DX), lambda i: (i, 0)))
    for m in range(4):
        in_specs.append(pl.BlockSpec((Lps[m], DX), lambda i: (0, 0)))
    for m in range(4):
        in_specs.append(pl.BlockSpec((1, DX), lambda i: (0, 0)))
    return pl.pallas_call(
        functools.partial(_k3_body, nchs, CWs),
        grid=(25,),
        in_specs=in_specs,
        out_specs=[pl.BlockSpec((rb, DX), lambda i: (i, 0))] * 5,
        out_shape=[jax.ShapeDtypeStruct((N, DX), F32)] * 5,
    )(*hparts, *xhats, *g1ws, *g1bs)


def _sct_body(rep, rv, rt, rp, rtr, unm, pim, nim,
              user_o, pos_o, neg_o, kpart,
              idxb, rows, kp_vm):
    cid = lax.axis_index("c")
    sid = lax.axis_index("s")
    wid = sid * NC + cid
    bpw = B // NW

    def gather_out(idx_hbm, out_hbm):
        pltpu.sync_copy(idx_hbm, idxb)
        pltpu.sync_copy(rep.at[idxb.at[wid]], rows)
        pltpu.sync_copy(rows, out_hbm.at[pl.ds(wid * bpw, bpw)])

    gather_out(unm, user_o)
    gather_out(nim, neg_o)
    gather_out(pim, pos_o)

    for m, tab in enumerate((rv, rt, rp, rtr)):
        pltpu.sync_copy(tab.at[idxb.at[wid]], rows)
        for j in range(DX // 16):
            def acc_body(e, a):
                return a + rows[e, pl.ds(j * 16, 16)]
            a = lax.fori_loop(0, bpw, acc_body, jnp.zeros((16,), F32),
                              unroll=8)
            kp_vm[m, pl.ds(j * 16, 16)] = a
    for m in range(4, 8):
        for j in range(DX // 16):
            kp_vm[m, pl.ds(j * 16, 16)] = jnp.zeros((16,), F32)
    pltpu.sync_copy(kp_vm, kpart.at[wid])


def _sct(rep, repms, unm, pim, nim):
    mesh = plsc.VectorSubcoreMesh(core_axis_name="c", subcore_axis_name="s", num_cores=NC, num_subcores=NS)
    bpw = B // NW
    return pl.kernel(
        _sct_body,
        out_type=[
            jax.ShapeDtypeStruct((B, DX), F32),
            jax.ShapeDtypeStruct((B, DX), F32),
            jax.ShapeDtypeStruct((B, DX), F32),
            jax.ShapeDtypeStruct((NW, 8, DX), F32),
        ],
        mesh=mesh,
        compiler_params=pltpu.CompilerParams(use_tc_tiling_on_sc=False, needs_layout_passes=False),
        scratch_types=[
            pltpu.VMEM((NW, bpw), I32),
            pltpu.VMEM((bpw, DX), F32),
            pltpu.VMEM((8, DX), F32),
        ],
    )(rep, *repms, unm, pim, nim)


def _ktail_body(u_ref, p_ref, n_ref, kp_ref, qt_ref, kt_ref, vt_ref,
                w1a_ref, w1b_ref, b1_ref, w2_ref, b2_ref,
                pos_ref, neg_ref, price_ref):
    K8 = jnp.sum(kp_ref[...], axis=0) * (1.0 / B)
    Kp = jnp.dot(K8, kt_ref[...], preferred_element_type=F32)
    Vp = jnp.dot(K8, vt_ref[...], preferred_element_type=F32)
    Q = jnp.dot(u_ref[...], qt_ref[...], preferred_element_type=F32)
    logits = lax.dot_general(Q, Kp, (((1,), (1,)), ((), ())),
                             preferred_element_type=F32) * (1.0 / 8.0)
    col = lax.broadcasted_iota(I32, (B, 8), 1)
    logits = jnp.where(col < 4, logits, -1e30)
    mx = jnp.max(logits, axis=1, keepdims=True)
    e = jnp.exp(logits - mx)
    att_w = e / jnp.sum(e, axis=1, keepdims=True)
    att = jnp.dot(att_w, Vp, preferred_element_type=F32)
    pos_t = p_ref[...]
    pos_ref[...] = jnp.sum(att * pos_t, axis=1, keepdims=True)
    neg_ref[...] = jnp.sum(att * n_ref[...], axis=1, keepdims=True)
    hid = _leaky(jnp.dot(att, w1a_ref[...], preferred_element_type=F32)
                 + jnp.dot(pos_t, w1b_ref[...], preferred_element_type=F32)
                 + b1_ref[...])
    pr = jnp.dot(hid, w2_ref[...], preferred_element_type=F32) + b2_ref[...]
    price_ref[...] = 1.0 / (1.0 + jnp.exp(-pr[:, 0:1]))


def _ktail(user_t, pos_t, neg_t, kpart, qT, kT, vT, w1a, w1b, b1, w2p, b2p):
    full = lambda *s: pl.BlockSpec(s, lambda: tuple(0 for _ in s))
    return pl.pallas_call(
        _ktail_body,
        in_specs=[
            full(B, DX), full(B, DX), full(B, DX), full(NW, 8, DX),
            full(DX, DX), full(DX, DX), full(DX, DX),
            full(DX, DX), full(DX, DX), full(1, DX),
            full(DX, 128), full(1, 128),
        ],
        out_specs=[full(B, 1), full(B, 1), full(B, 1)],
        out_shape=[
            jax.ShapeDtypeStruct((B, 1), F32),
            jax.ShapeDtypeStruct((B, 1), F32),
            jax.ShapeDtypeStruct((B, 1), F32),
        ],
    )(user_t, pos_t, neg_t, kpart, qT, kT, vT, w1a, w1b, b1, w2p, b2p)


def kernel(v_feat, t_feat, p_feat, tr_feat, user_feat, edge_index,
           v_mlp_w, v_mlp_b, v_umlp_w, v_umlp_b, v_conv_w, v_lin1_w,
           v_lin1_b, v_g1_w, v_g1_b,
           t_mlp_w, t_mlp_b, t_umlp_w, t_umlp_b, t_conv_w, t_lin1_w,
           t_lin1_b, t_g1_w, t_g1_b,
           p_mlp_w, p_mlp_b, p_umlp_w, p_umlp_b, p_conv_w, p_lin1_w,
           p_lin1_b, p_g1_w, p_g1_b,
           tr_mlp_w, tr_mlp_b, tr_umlp_w, tr_umlp_b, tr_conv_w, tr_lin1_w,
           tr_lin1_b, tr_g1_w, tr_g1_b,
           id_emb, q_w, k_w, v_w, price_w1, price_b1, price_w2, price_b2,
           user_nodes, pos_items, neg_items):
    mods = {
        "v": (v_feat, v_mlp_w, v_mlp_b, v_umlp_w, v_umlp_b, v_conv_w,
              v_lin1_w, v_lin1_b, v_g1_w, v_g1_b, 128, 1024, 1024, 128),
        "t": (t_feat, t_mlp_w, t_mlp_b, t_umlp_w, t_umlp_b, t_conv_w,
              t_lin1_w, t_lin1_b, t_g1_w, t_g1_b, 128, 1500, 1536, 128),
        "p": (p_feat, p_mlp_w, p_mlp_b, p_umlp_w, p_umlp_b, p_conv_w,
              p_lin1_w, p_lin1_b, p_g1_w, p_g1_b, 32, 64, 64, 64),
        "tr": (tr_feat, tr_mlp_w, tr_mlp_b, tr_umlp_w, tr_umlp_b, tr_conv_w,
               tr_lin1_w, tr_lin1_b, tr_g1_w, tr_g1_b, 32, 64, 64, 64),
    }

    src = edge_index[0]
    dst = edge_index[1]
    padi = jnp.zeros((EP - E,), I32)
    srcm = jnp.concatenate([src, padi]).reshape(NW * KB, 128)
    dstm = jnp.concatenate([dst, padi]).reshape(NW * KB, 128)
    padh = jnp.zeros((EPA - EH,), I32)
    srch = jnp.concatenate([src[:EH], padh]).reshape(NW * KBA, 128)
    dsth = jnp.concatenate([dst[:EH], padh]).reshape(NW * KBA, 128)

    hparts, xhats, g1ws, g1bs, nchs, CWs, Lps = [], [], [], [], [], [], []
    sparts = []
    scb_args = []
    for name, (feat, mw, mb, uw, ub, cw, lw, lb, g1w, g1b,
               Fm, L, Lp, CW) in mods.items():
        nch = Lp // 64
        x, xhat = _k1a(feat, user_feat, mw, mb.reshape(1, L), uw,
                       ub.reshape(1, L), lw, lb.reshape(1, DX), id_emb, L, Fm)
        convp = cw if Lp == L else jnp.pad(cw, ((0, 0), (0, Lp - L)))
        xw3, xw2 = _k1b(x, convp, L, Lp, CW)
        CW = 64
        spart, wout = _sca(xw2, srch, dsth, Lp)
        sparts.append(spart.reshape(NW, N))
        scb_args.append((xw3, wout, nch, CW))
        xhats.append(xhat)
        g1ws.append(g1w if Lp == L else jnp.pad(g1w, ((0, Lp - L), (0, 0))))
        g1bs.append(g1b.reshape(1, DX))
        nchs.append(nch)
        CWs.append(CW)
        Lps.append(Lp)

    s2s = _k2s(sparts)
    for (xw3, wout, nch, CW), s2 in zip(scb_args, s2s):
        hparts.append(_scb(xw3, srcm, dstm, wout.reshape(EPA),
                           s2, nch, CW))

    repm_v, repm_t, repm_p, repm_tr, rep = _k3(
        hparts, xhats, g1ws, g1bs, nchs, CWs, Lps)

    bpw = B // NW
    unm = user_nodes.reshape(NW, bpw)
    pim = pos_items.reshape(NW, bpw)
    nim = neg_items.reshape(NW, bpw)
    user_t, pos_t, neg_t, kpart = _sct(
        rep, (repm_v, repm_t, repm_p, repm_tr), unm, pim, nim)

    w2p = jnp.pad(price_w2, ((0, 0), (0, 127)))
    b2p = jnp.pad(price_b2.reshape(1, 1), ((0, 0), (0, 127)))
    pos_s, neg_s, price = _ktail(
        user_t, pos_t, neg_t, kpart, q_w.T, k_w.T, v_w.T,
        price_w1[:DX], price_w1[DX:], price_b1.reshape(1, DX), w2p, b2p)

    return (pos_s.reshape(B), neg_s.reshape(B), rep, price)


# speedup vs baseline: 1.4648x; 1.0032x over previous
import functools
import jax
import jax.numpy as jnp
from jax import lax
from jax.experimental import pallas as pl
from jax.experimental.pallas import tpu as pltpu
from jax.experimental.pallas import tpu_sc as plsc

N = 10000
NUM_ITEM = 8000
NUM_USER = 2000
E = 60000
B = 1024
DX = 64

NC = 2
NS = 16
NW = NC * NS
EW = 2048
EP = NW * EW
KB = 16
ECA = 32
NKA = EW // ECA
ZR = 80
NZC = N // ZR
EH = E // 2
EWA = 1024
EPA = NW * EWA
KBA = EWA // 128
NKA2 = EWA // ECA

F32 = jnp.float32
I32 = jnp.int32


def _leaky(x):
    return jnp.where(x >= 0, x, 0.01 * x)


def _k1a_body(feat_ref, user_ref, mw_ref, mb_ref, uw_ref, ub_ref,
              lw_ref, lb_ref, id_ref, x_ref, xhat_ref):
    pid = pl.program_id(0)

    @pl.when(pid < 8)
    def _():
        x_ref[...] = jnp.tanh(
            jnp.dot(feat_ref[...], mw_ref[...], preferred_element_type=F32)
            + mb_ref[...])

    @pl.when(pid >= 8)
    def _():
        x_ref[...] = jnp.tanh(
            jnp.dot(user_ref[...], uw_ref[...], preferred_element_type=F32)
            + ub_ref[...])

    r = x_ref[...]
    nrm = jnp.sqrt(jnp.sum(r * r, axis=1, keepdims=True))
    xl = r / jnp.maximum(nrm, 1e-12)
    x_ref[...] = xl
    xhat_ref[...] = _leaky(
        jnp.dot(xl, lw_ref[...], preferred_element_type=F32)
        + lb_ref[...]) + id_ref[...]


def _k1a(feat, user, mw, mb, uw, ub, lw, lb, id_emb, L, Fm):
    rb = 1000
    return pl.pallas_call(
        _k1a_body,
        grid=(10,),
        in_specs=[
            pl.BlockSpec((rb, Fm), lambda i: (jnp.minimum(i, 7), 0)),
            pl.BlockSpec((rb, 128), lambda i: (jnp.maximum(i - 8, 0), 0)),
            pl.BlockSpec((Fm, L), lambda i: (0, 0)),
            pl.BlockSpec((1, L), lambda i: (0, 0)),
            pl.BlockSpec((128, L), lambda i: (0, 0)),
            pl.BlockSpec((1, L), lambda i: (0, 0)),
            pl.BlockSpec((L, DX), lambda i: (0, 0)),
            pl.BlockSpec((1, DX), lambda i: (0, 0)),
            pl.BlockSpec((rb, DX), lambda i: (i, 0)),
        ],
        out_specs=[
            pl.BlockSpec((rb, L), lambda i: (i, 0)),
            pl.BlockSpec((rb, DX), lambda i: (i, 0)),
        ],
        out_shape=[
            jax.ShapeDtypeStruct((N, L), F32),
            jax.ShapeDtypeStruct((N, DX), F32),
        ],
    )(feat, user, mw, mb, uw, ub, lw, lb, id_emb)


def _k1b_body(nsub, x_ref, cw_ref, xw3_ref, xw2_ref):
    t = jnp.dot(x_ref[...], cw_ref[...], preferred_element_type=F32)
    for u in range(nsub):
        xw3_ref[u] = t[:, u * 64:(u + 1) * 64]
    xw2_ref[...] = t


def _k1b(x, convp, L, Lp, CWT):
    rb = 1000
    nch = Lp // CWT
    nsub = CWT // 64
    return pl.pallas_call(
        functools.partial(_k1b_body, nsub),
        grid=(10, nch),
        in_specs=[
            pl.BlockSpec((rb, L), lambda i, j: (i, 0)),
            pl.BlockSpec((L, CWT), lambda i, j: (0, j)),
        ],
        out_specs=[
            pl.BlockSpec((nsub, rb, 64), lambda i, j: (j, i, 0)),
            pl.BlockSpec((rb, CWT), lambda i, j: (i, j)),
        ],
        out_shape=[
            jax.ShapeDtypeStruct((Lp // 64, N, 64), F32),
            jax.ShapeDtypeStruct((N, Lp), F32),
        ],
    )(x, convp)


def _sca_body(Lp, xw2, srcm, dstm, spart, wout,
              idx_s, idx_d, rows_s, rows_d, wbuf, s_loc, gsem):
    cid = lax.axis_index("c")
    sid = lax.axis_index("s")
    wid = sid * NC + cid
    iota = lax.iota(I32, 16)

    pltpu.sync_copy(srcm.at[pl.ds(wid * KBA, KBA)], idx_s)
    pltpu.sync_copy(dstm.at[pl.ds(wid * KBA, KBA)], idx_d)

    def zero_body(i, _):
        s_loc[pl.ds(i * 16, 16)] = jnp.zeros((16,), F32)
        return 0
    lax.fori_loop(0, N // 16, zero_body, 0)

    def k_body(k, _):
        kb = k // 4
        off = (k % 4) * ECA
        cp1 = pltpu.async_copy(xw2.at[idx_s.at[kb, pl.ds(off, ECA)]],
                               rows_s, gsem)
        cp2 = pltpu.async_copy(xw2.at[idx_d.at[kb, pl.ds(off, ECA)]],
                               rows_d, gsem)
        cp1.wait()
        cp2.wait()

        def col_body(j, acc):
            a0, a1 = acc
            jv = jnp.full((16,), j, I32)
            s0 = plsc.load_gather(rows_s, [iota, jv])
            d0 = plsc.load_gather(rows_d, [iota, jv])
            s1 = plsc.load_gather(rows_s, [iota + 16, jv])
            d1 = plsc.load_gather(rows_d, [iota + 16, jv])
            return (a0 + s0 * d0, a1 + s1 * d1)

        z = jnp.zeros((16,), F32)
        a0, a1 = lax.fori_loop(0, Lp, col_body, (z, z), unroll=8)

        for g, inner in ((0, a0), (1, a1)):
            eid = wid * EWA + k * ECA + g * 16 + iota
            mask = eid < EH
            ev = jnp.where(mask, jnp.exp(inner), 0.0)
            gate = 1.0 / (1.0 + jnp.exp(-inner))
            w = jnp.where(mask, ev * gate, 0.0)
            wbuf[kb, pl.ds(off + g * 16, 16)] = w
            dst16 = idx_d[kb, pl.ds(off + g * 16, 16)]
            src16 = idx_s[kb, pl.ds(off + g * 16, 16)]
            plsc.addupdate_scatter(s_loc, [dst16], ev)
            plsc.addupdate_scatter(s_loc, [src16], ev)
        return 0

    lax.fori_loop(0, NKA2, k_body, 0)
    pltpu.sync_copy(s_loc, spart.at[pl.ds(wid * N, N)])
    pltpu.sync_copy(wbuf, wout.at[pl.ds(wid * KBA, KBA)])


def _sca(xw2, srcm, dstm, Lp):
    mesh = plsc.VectorSubcoreMesh(core_axis_name="c", subcore_axis_name="s", num_cores=NC, num_subcores=NS)
    return pl.kernel(
        functools.partial(_sca_body, Lp),
        out_type=[
            jax.ShapeDtypeStruct((NW * N,), F32),
            jax.ShapeDtypeStruct((NW * KBA, 128), F32),
        ],
        mesh=mesh,
        compiler_params=pltpu.CompilerParams(use_tc_tiling_on_sc=False, needs_layout_passes=False),
        scratch_types=[
            pltpu.VMEM((KBA, 128), I32),
            pltpu.VMEM((KBA, 128), I32),
            pltpu.VMEM((ECA, Lp), F32),
            pltpu.VMEM((ECA, Lp), F32),
            pltpu.VMEM((KBA, 128), F32),
            pltpu.VMEM((N,), F32),
            pltpu.SemaphoreType.DMA,
        ],
    )(xw2, srcm, dstm)


def _k2s_body(*refs):
    ins = refs[:4]
    outs = refs[4:]
    for i_ref, o_ref in zip(ins, outs):
        o_ref[...] = jnp.sum(i_ref[...], axis=0, keepdims=True)


def _k2s(sparts):
    return pl.pallas_call(
        _k2s_body,
        in_specs=[pl.BlockSpec((NW, N), lambda: (0, 0))] * 4,
        out_specs=[pl.BlockSpec((1, N), lambda: (0, 0))] * 4,
        out_shape=[jax.ShapeDtypeStruct((1, N), F32)] * 4,
    )(*sparts)


def _scb_body(nch, CW, xw3, srcm, dstm, wm, sm, hpart,
              idx_s, idx_d, wvm, alpha, s_vm, rows, scaled, zbuf, hsh,
              gsem, ssem):
    cid = lax.axis_index("c")
    sid = lax.axis_index("s")
    wid = sid * NC + cid

    iota = lax.iota(I32, 16)
    pltpu.sync_copy(srcm.at[pl.ds(wid * KB, KB)], idx_s)
    pltpu.sync_copy(dstm.at[pl.ds(wid * KB, KB)], idx_d)
    start1 = pl.multiple_of(jnp.minimum(wid * EW, EPA - EW), 8)
    start2 = pl.multiple_of(jnp.clip(wid * EW - EH, 0, EPA - EW), 8)
    pltpu.sync_copy(wm.at[pl.ds(start1, EW)], wvm.at[pl.ds(0, EW)])
    pltpu.sync_copy(wm.at[pl.ds(start2, EW)], wvm.at[pl.ds(EW, EW)])
    pltpu.sync_copy(sm.at[0], s_vm)

    def zb2_body(i, _):
        r = i // (CW // 16)
        c = (i % (CW // 16)) * 16
        zbuf[r, pl.ds(c, 16)] = jnp.zeros((16,), F32)
        return 0
    lax.fori_loop(0, ZR * (CW // 16), zb2_body, 0, unroll=8)

    def al_body(i, _):
        kb = i // 8
        off = (i % 8) * 16
        eid = wid * EW + i * 16 + iota
        mask = eid < E
        woff = jnp.where(eid < EH, eid - start1,
                         jnp.clip(eid - EH - start2, 0, EW - 1) + EW)
        woff = jnp.where(mask, woff, 0)
        dst16 = idx_d[kb, pl.ds(off, 16)]
        sv = plsc.load_gather(s_vm, [dst16])
        w16 = plsc.load_gather(wvm, [woff])
        a16 = jnp.where(mask, w16 / (sv + 1e-16), 0.0)
        alpha[kb, pl.ds(off, 16)] = a16
        return 0
    lax.fori_loop(0, KB * 8, al_body, 0)

    def chunk_body(c, _):
        def z_body(j, _):
            i = sid + j * NS

            @pl.when(i < NZC)
            def _():
                pltpu.sync_copy(zbuf, hsh.at[pl.ds(i * ZR, ZR)])
            return 0
        lax.fori_loop(0, (NZC + NS - 1) // NS, z_body, 0)
        plsc.subcore_barrier()

        def k4_body(kb4, _):
            cps = [
                pltpu.async_copy(xw3.at[c].at[idx_s.at[kb4 * 4 + u]],
                                 rows.at[pl.ds(u * 128, 128)], gsem)
                for u in range(4)
            ]
            for cp in cps:
                cp.wait()
            sps = []
            for u in range(4):
                kbg = kb4 * 4 + u
                if u >= 2:
                    sps[u - 2].wait()

                def e_body(e, _, u=u, kbg=kbg):
                    sp = plsc.load_gather(
                        alpha, [jnp.full((16,), kbg, I32),
                                jnp.full((16,), e, I32)])
                    for j in range(CW // 16):
                        scaled[(u % 2) * 128 + e, pl.ds(j * 16, 16)] = (
                            rows[u * 128 + e, pl.ds(j * 16, 16)] * sp)
                    return 0
                lax.fori_loop(0, 128, e_body, 0, unroll=8)
                sps.append(pltpu.async_copy(
                    scaled.at[pl.ds((u % 2) * 128, 128)],
                    hsh.at[idx_d.at[kbg]], ssem, add=True))
            sps[2].wait()
            sps[3].wait()
            return 0
        lax.fori_loop(0, KB // 4, k4_body, 0)
        plsc.subcore_barrier()

        def o_body(j, _):
            i = sid + j * NS

            @pl.when(i < NZC)
            def _():
                pltpu.sync_copy(hsh.at[pl.ds(i * ZR, ZR)],
                                hpart.at[cid, c, pl.ds(i * ZR, ZR)])
            return 0
        lax.fori_loop(0, (NZC + NS - 1) // NS, o_body, 0)
        plsc.subcore_barrier()
        return 0

    lax.fori_loop(0, nch, chunk_body, 0)


def _scb(xw3, srcm, dstm, wm, s2, nch, CW):
    mesh = plsc.VectorSubcoreMesh(core_axis_name="c", subcore_axis_name="s", num_cores=NC, num_subcores=NS)
    return pl.kernel(
        functools.partial(_scb_body, nch, CW),
        out_type=jax.ShapeDtypeStruct((NC, nch, N, CW), F32),
        mesh=mesh,
        compiler_params=pltpu.CompilerParams(use_tc_tiling_on_sc=False, needs_layout_passes=False),
        scratch_types=[
            pltpu.VMEM((KB, 128), I32),
            pltpu.VMEM((KB, 128), I32),
            pltpu.VMEM((2 * EW,), F32),
            pltpu.VMEM((KB, 128), F32),
            pltpu.VMEM((N,), F32),
            pltpu.VMEM((512, CW), F32),
            pltpu.VMEM((256, CW), F32),
            pltpu.VMEM((ZR, CW), F32),
            pltpu.VMEM_SHARED((N, CW), F32),
            pltpu.SemaphoreType.DMA,
            pltpu.SemaphoreType.DMA,
        ],
    )(xw3, srcm, dstm, wm, s2)


def _k3_body(nchs, CWs, *refs):
    hps = refs[0:4]
    xhs = refs[4:8]
    g1ws = refs[8:12]
    g1bs = refs[12:16]
    repms = refs[16:20]
    rep = refs[20]
    acc = None
    for m in range(4):
        hp = hps[m][...]
        h = hp[0] + hp[1]
        cw = CWs[m]
        mm = None
        for c in range(nchs[m]):
            part = jnp.dot(_leaky(h[c]),
                           g1ws[m][pl.ds(c * cw, cw), :],
                           preferred_element_type=F32)
            mm = part if mm is None else mm + part
        o = _leaky(mm + g1bs[m][...] + xhs[m][...])
        repms[m][...] = o
        acc = o if acc is None else acc + o
    rep[...] = acc * 0.25


def _k3(hparts, xhats, g1ws, g1bs, nchs, CWs, Lps):
    rb = 400
    in_specs = []
    for m in range(4):
        in_specs.append(pl.BlockSpec((NC, nchs[m], rb, CWs[m]),
                                     lambda i: (0, 0, i, 0)))
    for m in range(4):
        in_specs.append(pl.BlockSpec((rb, DX), lambda i: (i, 0)))
    for m in range(4):
        in_specs.append(pl.BlockSpec((Lps[m], DX), lambda i: (0, 0)))
    for m in range(4):
        in_specs.append(pl.BlockSpec((1, DX), lambda i: (0, 0)))
    return pl.pallas_call(
        functools.partial(_k3_body, nchs, CWs),
        grid=(25,),
        in_specs=in_specs,
        out_specs=[pl.BlockSpec((rb, DX), lambda i: (i, 0))] * 5,
        out_shape=[jax.ShapeDtypeStruct((N, DX), F32)] * 5,
    )(*hparts, *xhats, *g1ws, *g1bs)


def _sct_body(rep, rv, rt, rp, rtr, unm, pim, nim,
              user_o, pos_o, neg_o, kpart,
              idxb, rows, kp_vm):
    cid = lax.axis_index("c")
    sid = lax.axis_index("s")
    wid = sid * NC + cid
    bpw = B // NW

    def gather_out(idx_hbm, out_hbm):
        pltpu.sync_copy(idx_hbm, idxb)
        pltpu.sync_copy(rep.at[idxb.at[wid]], rows)
        pltpu.sync_copy(rows, out_hbm.at[pl.ds(wid * bpw, bpw)])

    gather_out(unm, user_o)
    gather_out(nim, neg_o)
    gather_out(pim, pos_o)

    for m, tab in enumerate((rv, rt, rp, rtr)):
        pltpu.sync_copy(tab.at[idxb.at[wid]], rows)
        for j in range(DX // 16):
            def acc_body(e, a):
                return a + rows[e, pl.ds(j * 16, 16)]
            a = lax.fori_loop(0, bpw, acc_body, jnp.zeros((16,), F32),
                              unroll=8)
            kp_vm[m, pl.ds(j * 16, 16)] = a
    for m in range(4, 8):
        for j in range(DX // 16):
            kp_vm[m, pl.ds(j * 16, 16)] = jnp.zeros((16,), F32)
    pltpu.sync_copy(kp_vm, kpart.at[wid])


def _sct(rep, repms, unm, pim, nim):
    mesh = plsc.VectorSubcoreMesh(core_axis_name="c", subcore_axis_name="s", num_cores=NC, num_subcores=NS)
    bpw = B // NW
    return pl.kernel(
        _sct_body,
        out_type=[
            jax.ShapeDtypeStruct((B, DX), F32),
            jax.ShapeDtypeStruct((B, DX), F32),
            jax.ShapeDtypeStruct((B, DX), F32),
            jax.ShapeDtypeStruct((NW, 8, DX), F32),
        ],
        mesh=mesh,
        compiler_params=pltpu.CompilerParams(use_tc_tiling_on_sc=False, needs_layout_passes=False),
        scratch_types=[
            pltpu.VMEM((NW, bpw), I32),
            pltpu.VMEM((bpw, DX), F32),
            pltpu.VMEM((8, DX), F32),
        ],
    )(rep, *repms, unm, pim, nim)


def _ktail_body(u_ref, p_ref, n_ref, kp_ref, qt_ref, kt_ref, vt_ref,
                w1a_ref, w1b_ref, b1_ref, w2_ref, b2_ref,
                pos_ref, neg_ref, price_ref):
    K8 = jnp.sum(kp_ref[...], axis=0) * (1.0 / B)
    Kp = jnp.dot(K8, kt_ref[...], preferred_element_type=F32)
    Vp = jnp.dot(K8, vt_ref[...], preferred_element_type=F32)
    Q = jnp.dot(u_ref[...], qt_ref[...], preferred_element_type=F32)
    logits = lax.dot_general(Q, Kp, (((1,), (1,)), ((), ())),
                             preferred_element_type=F32) * (1.0 / 8.0)
    col = lax.broadcasted_iota(I32, (B, 8), 1)
    logits = jnp.where(col < 4, logits, -1e30)
    mx = jnp.max(logits, axis=1, keepdims=True)
    e = jnp.exp(logits - mx)
    att_w = e / jnp.sum(e, axis=1, keepdims=True)
    att = jnp.dot(att_w, Vp, preferred_element_type=F32)
    pos_t = p_ref[...]
    pos_ref[...] = jnp.sum(att * pos_t, axis=1, keepdims=True)
    neg_ref[...] = jnp.sum(att * n_ref[...], axis=1, keepdims=True)
    hid = _leaky(jnp.dot(att, w1a_ref[...], preferred_element_type=F32)
                 + jnp.dot(pos_t, w1b_ref[...], preferred_element_type=F32)
                 + b1_ref[...])
    pr = jnp.dot(hid, w2_ref[...], preferred_element_type=F32) + b2_ref[...]
    price_ref[...] = 1.0 / (1.0 + jnp.exp(-pr[:, 0:1]))


def _ktail(user_t, pos_t, neg_t, kpart, qT, kT, vT, w1a, w1b, b1, w2p, b2p):
    full = lambda *s: pl.BlockSpec(s, lambda: tuple(0 for _ in s))
    return pl.pallas_call(
        _ktail_body,
        in_specs=[
            full(B, DX), full(B, DX), full(B, DX), full(NW, 8, DX),
            full(DX, DX), full(DX, DX), full(DX, DX),
            full(DX, DX), full(DX, DX), full(1, DX),
            full(DX, 128), full(1, 128),
        ],
        out_specs=[full(B, 1), full(B, 1), full(B, 1)],
        out_shape=[
            jax.ShapeDtypeStruct((B, 1), F32),
            jax.ShapeDtypeStruct((B, 1), F32),
            jax.ShapeDtypeStruct((B, 1), F32),
        ],
    )(user_t, pos_t, neg_t, kpart, qT, kT, vT, w1a, w1b, b1, w2p, b2p)


def kernel(v_feat, t_feat, p_feat, tr_feat, user_feat, edge_index,
           v_mlp_w, v_mlp_b, v_umlp_w, v_umlp_b, v_conv_w, v_lin1_w,
           v_lin1_b, v_g1_w, v_g1_b,
           t_mlp_w, t_mlp_b, t_umlp_w, t_umlp_b, t_conv_w, t_lin1_w,
           t_lin1_b, t_g1_w, t_g1_b,
           p_mlp_w, p_mlp_b, p_umlp_w, p_umlp_b, p_conv_w, p_lin1_w,
           p_lin1_b, p_g1_w, p_g1_b,
           tr_mlp_w, tr_mlp_b, tr_umlp_w, tr_umlp_b, tr_conv_w, tr_lin1_w,
           tr_lin1_b, tr_g1_w, tr_g1_b,
           id_emb, q_w, k_w, v_w, price_w1, price_b1, price_w2, price_b2,
           user_nodes, pos_items, neg_items):
    mods = {
        "v": (v_feat, v_mlp_w, v_mlp_b, v_umlp_w, v_umlp_b, v_conv_w,
              v_lin1_w, v_lin1_b, v_g1_w, v_g1_b, 128, 1024, 1024, 128),
        "t": (t_feat, t_mlp_w, t_mlp_b, t_umlp_w, t_umlp_b, t_conv_w,
              t_lin1_w, t_lin1_b, t_g1_w, t_g1_b, 128, 1500, 1536, 128),
        "p": (p_feat, p_mlp_w, p_mlp_b, p_umlp_w, p_umlp_b, p_conv_w,
              p_lin1_w, p_lin1_b, p_g1_w, p_g1_b, 32, 64, 64, 64),
        "tr": (tr_feat, tr_mlp_w, tr_mlp_b, tr_umlp_w, tr_umlp_b, tr_conv_w,
               tr_lin1_w, tr_lin1_b, tr_g1_w, tr_g1_b, 32, 64, 64, 64),
    }

    src = edge_index[0]
    dst = edge_index[1]
    padi = jnp.zeros((EP - E,), I32)
    srcm = jnp.concatenate([src, padi]).reshape(NW * KB, 128)
    dstm = jnp.concatenate([dst, padi]).reshape(NW * KB, 128)
    padh = jnp.zeros((EPA - EH,), I32)
    srch = jnp.concatenate([src[:EH], padh]).reshape(NW * KBA, 128)
    dsth = jnp.concatenate([dst[:EH], padh]).reshape(NW * KBA, 128)

    hparts, xhats, g1ws, g1bs, nchs, CWs, Lps = [], [], [], [], [], [], []
    sparts = []
    scb_args = []
    for name, (feat, mw, mb, uw, ub, cw, lw, lb, g1w, g1b,
               Fm, L, Lp, CW) in mods.items():
        nch = Lp // 64
        x, xhat = _k1a(feat, user_feat, mw, mb.reshape(1, L), uw,
                       ub.reshape(1, L), lw, lb.reshape(1, DX), id_emb, L, Fm)
        convp = cw if Lp == L else jnp.pad(cw, ((0, 0), (0, Lp - L)))
        xw3, xw2 = _k1b(x, convp, L, Lp, CW)
        CW = 64
        spart, wout = _sca(xw2, srch, dsth, Lp)
        sparts.append(spart.reshape(NW, N))
        scb_args.append((xw3, wout, nch, CW))
        xhats.append(xhat)
        g1ws.append(g1w if Lp == L else jnp.pad(g1w, ((0, Lp - L), (0, 0))))
        g1bs.append(g1b.reshape(1, DX))
        nchs.append(nch)
        CWs.append(CW)
        Lps.append(Lp)

    s2s = _k2s(sparts)
    for (xw3, wout, nch, CW), s2 in zip(scb_args, s2s):
        hparts.append(_scb(xw3, srcm, dstm, wout.reshape(EPA),
                           s2, nch, CW))

    repm_v, repm_t, repm_p, repm_tr, rep = _k3(
        hparts, xhats, g1ws, g1bs, nchs, CWs, Lps)

    bpw = B // NW
    unm = user_nodes.reshape(NW, bpw)
    pim = pos_items.reshape(NW, bpw)
    nim = neg_items.reshape(NW, bpw)
    user_t, pos_t, neg_t, kpart = _sct(
        rep, (repm_v, repm_t, repm_p, repm_tr), unm, pim, nim)

    w2p = jnp.pad(price_w2, ((0, 0), (0, 127)))
    b2p = jnp.pad(price_b2.reshape(1, 1), ((0, 0), (0, 127)))
    pos_s, neg_s, price = _ktail(
        user_t, pos_t, neg_t, kpart, q_w.T, k_w.T, v_w.T,
        price_w1[:DX], price_w1[DX:], price_b1.reshape(1, DX), w2p, b2p)

    return (pos_s.reshape(B), neg_s.reshape(B), rep, price)
